# Initial kernel scaffold; baseline (speedup 1.0000x reference)
import jax
import jax.numpy as jnp
from jax.experimental import pallas as pl


def kernel(item, uh_edge_index, v_uh_edge_index, feature_video_mapping, features, u_h_embedding, W_tv, b_tv, W_g1, a1_src, a1_dst, W_s2l, W_s2r, W_g3, a3_src, a3_dst, W_s4l, W_s4r, W_uv, b_uv, W_uh, b_uh):
    raise NotImplementedError("write your pallas kernel here")


# speedup vs baseline: 8.4225x; 8.4225x over previous
import functools

import jax
import jax.numpy as jnp
from jax import lax
from jax.experimental import pallas as pl
from jax.experimental.pallas import tpu as pltpu
from jax.experimental.pallas import tpu_sc as plsc

NU, NH, NV = 4000, 1000, 5000
NUH = NU + NH
N = NUH + NV
D, DF, E, B = 128, 512, 320000, 1024

NC, NS = 2, 16
NW = NC * NS
CH = 64
NCHUNK = E // CH
FULL = NCHUNK // NW
EXTRA = NCHUNK - FULL * NW
RPT = 624
DEN_W = 16

_HI = lax.Precision.HIGHEST


def _lrelu(x, s):
    return jnp.where(x > 0, x, x * s)


def _feat_body(feat, uh, Wtv, btv, xv_o, x0_o):
    xv = _lrelu(jnp.dot(feat[...], Wtv[...], precision=_HI) + btv[...], 0.01)
    xv_o[...] = xv
    x = jnp.concatenate([uh[...], xv], axis=0)
    nrm = jnp.sqrt(jnp.sum(x * x, axis=1, keepdims=True))
    x0_o[...] = x / jnp.maximum(nrm, 1e-12)


_feat_k = pl.pallas_call(
    _feat_body,
    out_shape=[jax.ShapeDtypeStruct((NV, D), jnp.float32),
               jax.ShapeDtypeStruct((N, D), jnp.float32)])


def _prep_body(x, Wg, a_s, a_d, h_o, hs_o, hd_o, gm_o):
    i = pl.program_id(0)
    h = jnp.dot(x[...], Wg[...], precision=_HI)
    h_o[...] = h
    hs = jnp.dot(h, a_s[...], precision=_HI)
    hd = jnp.dot(h, a_d[...], precision=_HI)
    hs_o[...] = hs
    hd_o[...] = hd

    @pl.when(i == 0)
    def _():
        gm_o[...] = jnp.full((1, 1), -jnp.inf)

    gm_o[...] = jnp.maximum(gm_o[...], jnp.max(hs))


_RB = 2000

_prep_k = pl.pallas_call(
    _prep_body,
    grid=(N // _RB,),
    in_specs=[pl.BlockSpec((_RB, D), lambda i: (i, 0)),
              pl.BlockSpec((D, D), lambda i: (0, 0)),
              pl.BlockSpec((D, 1), lambda i: (0, 0)),
              pl.BlockSpec((D, 1), lambda i: (0, 0))],
    out_specs=[pl.BlockSpec((_RB, D), lambda i: (i, 0)),
               pl.BlockSpec((_RB, 1), lambda i: (i, 0)),
               pl.BlockSpec((_RB, 1), lambda i: (i, 0)),
               pl.BlockSpec((1, 1), lambda i: (0, 0))],
    out_shape=[jax.ShapeDtypeStruct((N, D), jnp.float32),
               jax.ShapeDtypeStruct((N, 1), jnp.float32),
               jax.ShapeDtypeStruct((N, 1), jnp.float32),
               jax.ShapeDtypeStruct((1, 1), jnp.float32)])


def _postgat_body(acc0, acc1, den0, den1, x_o):
    a = acc0[...] + acc1[...]
    d = den0[:, 0] + den1[:, 0]
    x_o[...] = _lrelu(a / (d[:, None] + 1e-16), 0.01)


_postgat_k = pl.pallas_call(
    _postgat_body,
    grid=(N // _RB,),
    in_specs=[pl.BlockSpec((_RB, D), lambda i: (i, 0)),
              pl.BlockSpec((_RB, D), lambda i: (i, 0)),
              pl.BlockSpec((_RB, DEN_W), lambda i: (i, 0)),
              pl.BlockSpec((_RB, DEN_W), lambda i: (i, 0))],
    out_specs=pl.BlockSpec((_RB, D), lambda i: (i, 0)),
    out_shape=jax.ShapeDtypeStruct((N, D), jnp.float32))


def _postsage_body(sacc0, sacc1, cnt0, cnt1, x, Wl, Wr, x_o):
    s = sacc0[...] + sacc1[...]
    c = cnt0[:, 0] + cnt1[:, 0]
    mean = s / jnp.maximum(c, 1.0)[:, None]
    x_o[...] = _lrelu(jnp.dot(mean, Wl[...], precision=_HI)
                      + jnp.dot(x[...], Wr[...], precision=_HI), 0.01)


_postsage_k = pl.pallas_call(
    _postsage_body,
    grid=(N // _RB,),
    in_specs=[pl.BlockSpec((_RB, D), lambda i: (i, 0)),
              pl.BlockSpec((_RB, D), lambda i: (i, 0)),
              pl.BlockSpec((_RB, DEN_W), lambda i: (i, 0)),
              pl.BlockSpec((_RB, DEN_W), lambda i: (i, 0)),
              pl.BlockSpec((_RB, D), lambda i: (i, 0)),
              pl.BlockSpec((D, D), lambda i: (0, 0)),
              pl.BlockSpec((D, D), lambda i: (0, 0))],
    out_specs=pl.BlockSpec((_RB, D), lambda i: (i, 0)),
    out_shape=jax.ShapeDtypeStruct((N, D), jnp.float32))


def _score_body(user, pos, vid, Wuv, buv, Wuh, buh, out_o):
    usv = _lrelu(jnp.dot(jnp.concatenate([vid[...], user[...]], axis=1),
                         Wuv[...], precision=_HI) + buv[...], 0.01)
    usp = _lrelu(jnp.dot(jnp.concatenate([pos[...], user[...]], axis=1),
                         Wuh[...], precision=_HI) + buh[...], 0.01)
    out_o[...] = jnp.sum(usv * usp, axis=1, keepdims=True)


_score_k = pl.pallas_call(
    _score_body,
    out_shape=jax.ShapeDtypeStruct((B, 1), jnp.float32))


_sc_mesh = plsc.VectorSubcoreMesh(core_axis_name="c", subcore_axis_name="s")
_Z16F = functools.partial(jnp.zeros, (16,), jnp.float32)


def _init_shared(src_ref, sh_ref, row0, s):
    for j in range(RPT // CH):
        pltpu.sync_copy(src_ref, sh_ref.at[pl.ds(row0 + j * CH, CH)])
    rem = RPT - (RPT // CH) * CH
    if rem:
        pltpu.sync_copy(src_ref.at[pl.ds(0, rem)],
                        sh_ref.at[pl.ds(row0 + (RPT // CH) * CH, rem)])

    @pl.when(s == NS - 1)
    def _():
        pltpu.sync_copy(src_ref.at[pl.ds(0, N - NS * RPT)],
                        sh_ref.at[pl.ds(NS * RPT, N - NS * RPT)])


def _copy_out(sh_ref, out0_ref, out1_ref, c, row0, s):
    def emit(out_ref):
        pltpu.sync_copy(sh_ref.at[pl.ds(row0, RPT)],
                        out_ref.at[pl.ds(row0, RPT)])

        @pl.when(s == NS - 1)
        def _():
            pltpu.sync_copy(sh_ref.at[pl.ds(NS * RPT, N - NS * RPT)],
                            out_ref.at[pl.ds(NS * RPT, N - NS * RPT)])

    @pl.when(c == 0)
    def _():
        emit(out0_ref)

    @pl.when(c == 1)
    def _():
        emit(out1_ref)


def _gat_edge(h_hbm, hs_hbm, hd_hbm, g16_hbm, zr_hbm, z16_hbm,
              src_hbm, dst_hbm,
              acc0_o, acc1_o, den0_o, den1_o,
              hs_v, hd_v, g16_v, srcA, dstA, srcB, dstB, w_v,
              rowsA, rowsB, w16_v, acc_sh, den_sh, sem):
    c = lax.axis_index("c")
    s = lax.axis_index("s")
    wid = s * NC + c
    row0 = s * RPT
    nchunk = FULL + jnp.where(wid < EXTRA, 1, 0)

    pltpu.sync_copy(hs_hbm, hs_v)
    pltpu.sync_copy(hd_hbm, hd_v)
    pltpu.sync_copy(g16_hbm, g16_v)
    pltpu.sync_copy(zr_hbm, rowsA)
    pltpu.sync_copy(z16_hbm, w16_v)
    _init_shared(rowsA, acc_sh, row0, s)
    _init_shared(w16_v, den_sh, row0, s)
    plsc.subcore_barrier()
    g16 = g16_v[pl.ds(0, 16)]

    def cid(q):
        return jnp.where(q < FULL, wid + q * NW, FULL * NW + wid)

    def fetch_idx(q, src_v, dst_v):
        base = cid(q) * CH
        pltpu.sync_copy(src_hbm.at[pl.ds(base, CH)], src_v)
        pltpu.sync_copy(dst_hbm.at[pl.ds(base, CH)], dst_v)

    def wloop(src_v, dst_v):
        for k in range(CH // 16):
            sv = src_v[pl.ds(k * 16, 16)]
            dv = dst_v[pl.ds(k * 16, 16)]
            hsg = plsc.load_gather(hs_v, [sv])
            hdg = plsc.load_gather(hd_v, [dv])
            mg = g16 + hdg
            mg = jnp.where(mg > 0, mg, mg * 0.2)
            a = hsg + hdg
            a = jnp.where(a > 0, a, a * 0.2)
            w_v[pl.ds(k * 16, 16)] = jnp.exp(a - mg)

    def srow_scatter(rows_v, dst_v):
        for r in range(CH):
            wb = plsc.load_gather(w_v, [jnp.full((16,), r, jnp.int32)])
            w16_v[r, pl.ds(0, 16)] = wb
            for k in range(D // 16):
                rows_v[r, pl.ds(k * 16, 16)] = (
                    rows_v[r, pl.ds(k * 16, 16)] * wb)
        pltpu.sync_copy(rows_v, acc_sh.at[dst_v], add=True)
        pltpu.sync_copy(w16_v, den_sh.at[dst_v], add=True)

    fetch_idx(0, srcA, dstA)
    pltpu.async_copy(h_hbm.at[srcA], rowsA, sem)

    def half(q, src_v, dst_v, rows_v, nsrc_v, ndst_v, nrows_v):
        @pl.when(q < nchunk)
        def _():
            wloop(src_v, dst_v)
            pltpu.make_async_copy(h_hbm.at[src_v], rows_v, sem).wait()

            @pl.when(q + 1 < nchunk)
            def _():
                fetch_idx(q + 1, nsrc_v, ndst_v)
                pltpu.async_copy(h_hbm.at[nsrc_v], nrows_v, sem)

            srow_scatter(rows_v, dst_v)

    def pair(p, _):
        half(2 * p, srcA, dstA, rowsA, srcB, dstB, rowsB)
        half(2 * p + 1, srcB, dstB, rowsB, srcA, dstA, rowsA)
        return 0

    lax.fori_loop(0, (FULL + 2) // 2, pair, 0)

    plsc.subcore_barrier()
    _copy_out(acc_sh, acc0_o, acc1_o, c, row0, s)
    _copy_out(den_sh, den0_o, den1_o, c, row0, s)


_gat_edge_k = pl.kernel(
    _gat_edge,
    out_type=[jax.ShapeDtypeStruct((N, D), jnp.float32),
              jax.ShapeDtypeStruct((N, D), jnp.float32),
              jax.ShapeDtypeStruct((N, DEN_W), jnp.float32),
              jax.ShapeDtypeStruct((N, DEN_W), jnp.float32)],
    mesh=_sc_mesh,
    compiler_params=pltpu.CompilerParams(needs_layout_passes=False, use_tc_tiling_on_sc=False),
    scratch_types=[
        pltpu.VMEM((N,), jnp.float32),
        pltpu.VMEM((N,), jnp.float32),
        pltpu.VMEM((16,), jnp.float32),
        pltpu.VMEM((CH,), jnp.int32),
        pltpu.VMEM((CH,), jnp.int32),
        pltpu.VMEM((CH,), jnp.int32),
        pltpu.VMEM((CH,), jnp.int32),
        pltpu.VMEM((CH,), jnp.float32),
        pltpu.VMEM((CH, D), jnp.float32),
        pltpu.VMEM((CH, D), jnp.float32),
        pltpu.VMEM((CH, DEN_W), jnp.float32),
        pltpu.VMEM_SHARED((N, D), jnp.float32),
        pltpu.VMEM_SHARED((N, DEN_W), jnp.float32),
        pltpu.SemaphoreType.DMA,
    ])


def _sage_edge(with_cnt, *refs):
    if with_cnt:
        (x_hbm, zr_hbm, z16_hbm, ones_hbm, src_hbm, dst_hbm,
         sacc0_o, sacc1_o, cnt0_o, cnt1_o,
         srcA, dstA, srcB, dstB, rowsA, rowsB, ones_v,
         acc_sh, cnt_sh, sem) = refs
    else:
        (x_hbm, zr_hbm, src_hbm, dst_hbm,
         sacc0_o, sacc1_o,
         srcA, dstA, srcB, dstB, rowsA, rowsB,
         acc_sh, sem) = refs
    c = lax.axis_index("c")
    s = lax.axis_index("s")
    wid = s * NC + c
    row0 = s * RPT
    nchunk = FULL + jnp.where(wid < EXTRA, 1, 0)

    pltpu.sync_copy(zr_hbm, rowsA)
    _init_shared(rowsA, acc_sh, row0, s)
    if with_cnt:
        pltpu.sync_copy(z16_hbm, ones_v)
        _init_shared(ones_v, cnt_sh, row0, s)
        pltpu.sync_copy(ones_hbm, ones_v)
    plsc.subcore_barrier()

    def cid(q):
        return jnp.where(q < FULL, wid + q * NW, FULL * NW + wid)

    def fetch_idx(q, src_v, dst_v):
        base = cid(q) * CH
        pltpu.sync_copy(src_hbm.at[pl.ds(base, CH)], src_v)
        pltpu.sync_copy(dst_hbm.at[pl.ds(base, CH)], dst_v)

    fetch_idx(0, srcA, dstA)
    pltpu.async_copy(x_hbm.at[srcA], rowsA, sem)

    def half(q, src_v, dst_v, rows_v, nsrc_v, ndst_v, nrows_v):
        @pl.when(q < nchunk)
        def _():
            pltpu.make_async_copy(x_hbm.at[src_v], rows_v, sem).wait()

            @pl.when(q + 1 < nchunk)
            def _():
                fetch_idx(q + 1, nsrc_v, ndst_v)
                pltpu.async_copy(x_hbm.at[nsrc_v], nrows_v, sem)

            pltpu.sync_copy(rows_v, acc_sh.at[dst_v], add=True)
            if with_cnt:
                pltpu.sync_copy(ones_v, cnt_sh.at[dst_v], add=True)

    def pair(p, _):
        half(2 * p, srcA, dstA, rowsA, srcB, dstB, rowsB)
        half(2 * p + 1, srcB, dstB, rowsB, srcA, dstA, rowsA)
        return 0

    lax.fori_loop(0, (FULL + 2) // 2, pair, 0)

    plsc.subcore_barrier()
    _copy_out(acc_sh, sacc0_o, sacc1_o, c, row0, s)
    if with_cnt:
        _copy_out(cnt_sh, cnt0_o, cnt1_o, c, row0, s)


_sc_params = pltpu.CompilerParams(needs_layout_passes=False,
                                  use_tc_tiling_on_sc=False)

_sage_cnt_k = pl.kernel(
    functools.partial(_sage_edge, True),
    out_type=[jax.ShapeDtypeStruct((N, D), jnp.float32),
              jax.ShapeDtypeStruct((N, D), jnp.float32),
              jax.ShapeDtypeStruct((N, DEN_W), jnp.float32),
              jax.ShapeDtypeStruct((N, DEN_W), jnp.float32)],
    mesh=_sc_mesh,
    compiler_params=_sc_params,
    scratch_types=[
        pltpu.VMEM((CH,), jnp.int32),
        pltpu.VMEM((CH,), jnp.int32),
        pltpu.VMEM((CH,), jnp.int32),
        pltpu.VMEM((CH,), jnp.int32),
        pltpu.VMEM((CH, D), jnp.float32),
        pltpu.VMEM((CH, D), jnp.float32),
        pltpu.VMEM((CH, DEN_W), jnp.float32),
        pltpu.VMEM_SHARED((N, D), jnp.float32),
        pltpu.VMEM_SHARED((N, DEN_W), jnp.float32),
        pltpu.SemaphoreType.DMA,
    ])

_sage_nocnt_k = pl.kernel(
    functools.partial(_sage_edge, False),
    out_type=[jax.ShapeDtypeStruct((N, D), jnp.float32),
              jax.ShapeDtypeStruct((N, D), jnp.float32)],
    mesh=_sc_mesh,
    compiler_params=_sc_params,
    scratch_types=[
        pltpu.VMEM((CH,), jnp.int32),
        pltpu.VMEM((CH,), jnp.int32),
        pltpu.VMEM((CH,), jnp.int32),
        pltpu.VMEM((CH,), jnp.int32),
        pltpu.VMEM((CH, D), jnp.float32),
        pltpu.VMEM((CH, D), jnp.float32),
        pltpu.VMEM_SHARED((N, D), jnp.float32),
        pltpu.SemaphoreType.DMA,
    ])

_BPW = B // NW


def _readout(x4_hbm, xv_hbm, fvm_hbm, it0_hbm, it1_hbm, it2_hbm,
             user_o, pos_o, vid_o,
             fvm_v, i0_v, i1_v, i2_v, vi_v, u_v, p_v, v_v, sem):
    c = lax.axis_index("c")
    s = lax.axis_index("s")
    wid = s * NC + c
    base = wid * _BPW
    pltpu.sync_copy(fvm_hbm, fvm_v)
    pltpu.sync_copy(it0_hbm.at[pl.ds(base, _BPW)], i0_v)
    pltpu.sync_copy(it1_hbm.at[pl.ds(base, _BPW)], i1_v)
    pltpu.sync_copy(it2_hbm.at[pl.ds(base, _BPW)], i2_v)
    for k in range(_BPW // 16):
        iv = i1_v[pl.ds(k * 16, 16)]
        vi_v[pl.ds(k * 16, 16)] = plsc.load_gather(fvm_v, [iv])
    pltpu.async_copy(x4_hbm.at[i0_v], u_v, sem).wait()
    pltpu.async_copy(x4_hbm.at[i2_v], p_v, sem).wait()
    pltpu.async_copy(xv_hbm.at[vi_v], v_v, sem).wait()
    pltpu.sync_copy(u_v, user_o.at[pl.ds(base, _BPW)])
    pltpu.sync_copy(p_v, pos_o.at[pl.ds(base, _BPW)])
    pltpu.sync_copy(v_v, vid_o.at[pl.ds(base, _BPW)])


_readout_k = pl.kernel(
    _readout,
    out_type=[jax.ShapeDtypeStruct((B, D), jnp.float32),
              jax.ShapeDtypeStruct((B, D), jnp.float32),
              jax.ShapeDtypeStruct((B, D), jnp.float32)],
    mesh=_sc_mesh,
    compiler_params=pltpu.CompilerParams(needs_layout_passes=False, use_tc_tiling_on_sc=False),
    scratch_types=[
        pltpu.VMEM((NV,), jnp.int32),
        pltpu.VMEM((_BPW,), jnp.int32),
        pltpu.VMEM((_BPW,), jnp.int32),
        pltpu.VMEM((_BPW,), jnp.int32),
        pltpu.VMEM((_BPW,), jnp.int32),
        pltpu.VMEM((_BPW, D), jnp.float32),
        pltpu.VMEM((_BPW, D), jnp.float32),
        pltpu.VMEM((_BPW, D), jnp.float32),
        pltpu.SemaphoreType.DMA,
    ])


def kernel(item, uh_edge_index, v_uh_edge_index, feature_video_mapping,
           features, u_h_embedding, W_tv, b_tv, W_g1, a1_src, a1_dst,
           W_s2l, W_s2r, W_g3, a3_src, a3_dst, W_s4l, W_s4r,
           W_uv, b_uv, W_uh, b_uh):
    src1 = v_uh_edge_index[0]
    dst1 = v_uh_edge_index[1]
    src2 = uh_edge_index[0]
    dst2 = uh_edge_index[1]
    it0 = item[:, 0].astype(jnp.int32)
    it1 = item[:, 1].astype(jnp.int32)
    it2 = item[:, 2].astype(jnp.int32)

    xv, x0 = _feat_k(features, u_h_embedding, W_tv, b_tv[None, :])

    h1, hs1, hd1, gm1 = _prep_k(x0, W_g1, a1_src[:, None], a1_dst[:, None])
    zr = jnp.zeros((CH, D), jnp.float32)
    z16 = jnp.zeros((CH, DEN_W), jnp.float32)
    ones16 = jnp.ones((CH, DEN_W), jnp.float32)
    ga0, ga1, gd0, gd1 = _gat_edge_k(h1, hs1.reshape(N), hd1.reshape(N),
                                     jnp.broadcast_to(gm1.reshape(1), (16,)),
                                     zr, z16, src1, dst1)
    x1 = _postgat_k(ga0, ga1, gd0, gd1)

    sa0, sa1, sc0, sc1 = _sage_cnt_k(x1, zr, z16, ones16, src2, dst2)
    x2 = _postsage_k(sa0, sa1, sc0, sc1, x1, W_s2l, W_s2r)

    h3, hs3, hd3, gm3 = _prep_k(x2, W_g3, a3_src[:, None], a3_dst[:, None])
    gb0, gb1, ge0, ge1 = _gat_edge_k(h3, hs3.reshape(N), hd3.reshape(N),
                                     jnp.broadcast_to(gm3.reshape(1), (16,)),
                                     zr, z16, src1, dst1)
    x3 = _postgat_k(gb0, gb1, ge0, ge1)

    sb0, sb1 = _sage_nocnt_k(x3, zr, src2, dst2)
    x4 = _postsage_k(sb0, sb1, sc0, sc1, x3, W_s4l, W_s4r)

    user, pos, vid = _readout_k(x4, xv, feature_video_mapping,
                                it0, it1, it2)
    scores = _score_k(user, pos, vid, W_uv, b_uv[None, :],
                      W_uh, b_uh[None, :])
    return scores.reshape(B)


# speedup vs baseline: 8.7342x; 1.0370x over previous
import functools

import jax
import jax.numpy as jnp
from jax import lax
from jax.experimental import pallas as pl
from jax.experimental.pallas import tpu as pltpu
from jax.experimental.pallas import tpu_sc as plsc

NU, NH, NV = 4000, 1000, 5000
NUH = NU + NH
N = NUH + NV
D, DF, E, B = 128, 512, 320000, 1024

NC, NS = 2, 16
NW = NC * NS
CH = 64
NCHUNK = E // CH
FULL = NCHUNK // NW
EXTRA = NCHUNK - FULL * NW
RPT = 624
DEN_W = 16

_HI = lax.Precision.HIGHEST


def _lrelu(x, s):
    return jnp.where(x > 0, x, x * s)


def _feat_body(feat, uh, Wtv, btv, xv_o, x0_o):
    xv = _lrelu(jnp.dot(feat[...], Wtv[...], precision=_HI) + btv[...], 0.01)
    xv_o[...] = xv
    x = jnp.concatenate([uh[...], xv], axis=0)
    nrm = jnp.sqrt(jnp.sum(x * x, axis=1, keepdims=True))
    x0_o[...] = x / jnp.maximum(nrm, 1e-12)


_feat_k = pl.pallas_call(
    _feat_body,
    out_shape=[jax.ShapeDtypeStruct((NV, D), jnp.float32),
               jax.ShapeDtypeStruct((N, D), jnp.float32)])


def _prep_body(x, Wg, a_s, a_d, h_o, hs_o, hd_o, gm_o):
    i = pl.program_id(0)
    h = jnp.dot(x[...], Wg[...], precision=_HI)
    h_o[...] = h
    hs = jnp.dot(h, a_s[...], precision=_HI)
    hd = jnp.dot(h, a_d[...], precision=_HI)
    hs_o[...] = hs
    hd_o[...] = hd

    @pl.when(i == 0)
    def _():
        gm_o[...] = jnp.full((1, 1), -jnp.inf)

    gm_o[...] = jnp.maximum(gm_o[...], jnp.max(hs))


_RB = 2000

_prep_k = pl.pallas_call(
    _prep_body,
    grid=(N // _RB,),
    in_specs=[pl.BlockSpec((_RB, D), lambda i: (i, 0)),
              pl.BlockSpec((D, D), lambda i: (0, 0)),
              pl.BlockSpec((D, 1), lambda i: (0, 0)),
              pl.BlockSpec((D, 1), lambda i: (0, 0))],
    out_specs=[pl.BlockSpec((_RB, D), lambda i: (i, 0)),
               pl.BlockSpec((_RB, 1), lambda i: (i, 0)),
               pl.BlockSpec((_RB, 1), lambda i: (i, 0)),
               pl.BlockSpec((1, 1), lambda i: (0, 0))],
    out_shape=[jax.ShapeDtypeStruct((N, D), jnp.float32),
               jax.ShapeDtypeStruct((N, 1), jnp.float32),
               jax.ShapeDtypeStruct((N, 1), jnp.float32),
               jax.ShapeDtypeStruct((1, 1), jnp.float32)])


def _postgat_body(acc0, acc1, den0, den1, x_o):
    a = acc0[...] + acc1[...]
    d = den0[:, 0] + den1[:, 0]
    x_o[...] = _lrelu(a / (d[:, None] + 1e-16), 0.01)


_postgat_k = pl.pallas_call(
    _postgat_body,
    grid=(N // _RB,),
    in_specs=[pl.BlockSpec((_RB, D), lambda i: (i, 0)),
              pl.BlockSpec((_RB, D), lambda i: (i, 0)),
              pl.BlockSpec((_RB, DEN_W), lambda i: (i, 0)),
              pl.BlockSpec((_RB, DEN_W), lambda i: (i, 0))],
    out_specs=pl.BlockSpec((_RB, D), lambda i: (i, 0)),
    out_shape=jax.ShapeDtypeStruct((N, D), jnp.float32))


def _postsage_body(sacc0, sacc1, cnt0, cnt1, x, Wl, Wr, x_o):
    s = sacc0[...] + sacc1[...]
    c = cnt0[:, 0] + cnt1[:, 0]
    mean = s / jnp.maximum(c, 1.0)[:, None]
    x_o[...] = _lrelu(jnp.dot(mean, Wl[...], precision=_HI)
                      + jnp.dot(x[...], Wr[...], precision=_HI), 0.01)


_postsage_k = pl.pallas_call(
    _postsage_body,
    grid=(N // _RB,),
    in_specs=[pl.BlockSpec((_RB, D), lambda i: (i, 0)),
              pl.BlockSpec((_RB, D), lambda i: (i, 0)),
              pl.BlockSpec((_RB, DEN_W), lambda i: (i, 0)),
              pl.BlockSpec((_RB, DEN_W), lambda i: (i, 0)),
              pl.BlockSpec((_RB, D), lambda i: (i, 0)),
              pl.BlockSpec((D, D), lambda i: (0, 0)),
              pl.BlockSpec((D, D), lambda i: (0, 0))],
    out_specs=pl.BlockSpec((_RB, D), lambda i: (i, 0)),
    out_shape=jax.ShapeDtypeStruct((N, D), jnp.float32))


def _score_body(user, pos, vid, Wuv, buv, Wuh, buh, out_o):
    usv = _lrelu(jnp.dot(jnp.concatenate([vid[...], user[...]], axis=1),
                         Wuv[...], precision=_HI) + buv[...], 0.01)
    usp = _lrelu(jnp.dot(jnp.concatenate([pos[...], user[...]], axis=1),
                         Wuh[...], precision=_HI) + buh[...], 0.01)
    out_o[...] = jnp.sum(usv * usp, axis=1, keepdims=True)


_score_k = pl.pallas_call(
    _score_body,
    out_shape=jax.ShapeDtypeStruct((B, 1), jnp.float32))


_sc_mesh = plsc.VectorSubcoreMesh(core_axis_name="c", subcore_axis_name="s")
_Z16F = functools.partial(jnp.zeros, (16,), jnp.float32)


def _init_shared(src_ref, sh_ref, row0, s):
    for j in range(RPT // CH):
        pltpu.sync_copy(src_ref, sh_ref.at[pl.ds(row0 + j * CH, CH)])
    rem = RPT - (RPT // CH) * CH
    if rem:
        pltpu.sync_copy(src_ref.at[pl.ds(0, rem)],
                        sh_ref.at[pl.ds(row0 + (RPT // CH) * CH, rem)])

    @pl.when(s == NS - 1)
    def _():
        pltpu.sync_copy(src_ref.at[pl.ds(0, N - NS * RPT)],
                        sh_ref.at[pl.ds(NS * RPT, N - NS * RPT)])


def _copy_out(sh_ref, out0_ref, out1_ref, c, row0, s):
    def emit(out_ref):
        pltpu.sync_copy(sh_ref.at[pl.ds(row0, RPT)],
                        out_ref.at[pl.ds(row0, RPT)])

        @pl.when(s == NS - 1)
        def _():
            pltpu.sync_copy(sh_ref.at[pl.ds(NS * RPT, N - NS * RPT)],
                            out_ref.at[pl.ds(NS * RPT, N - NS * RPT)])

    @pl.when(c == 0)
    def _():
        emit(out0_ref)

    @pl.when(c == 1)
    def _():
        emit(out1_ref)


def _gat_edge(h_hbm, hs_hbm, hd_hbm, g16_hbm, zr_hbm, z16_hbm,
              src_hbm, dst_hbm,
              acc0_o, acc1_o, den0_o, den1_o,
              hs_v, hd_v, g16_v, srcA, dstA, srcB, dstB, w_v,
              rowsA, rowsB, w16_v, acc_sh, den_sh, sem, sem_s):
    c = lax.axis_index("c")
    s = lax.axis_index("s")
    wid = s * NC + c
    row0 = s * RPT
    nchunk = FULL + jnp.where(wid < EXTRA, 1, 0)

    pltpu.sync_copy(hs_hbm, hs_v)
    pltpu.sync_copy(hd_hbm, hd_v)
    pltpu.sync_copy(g16_hbm, g16_v)
    pltpu.sync_copy(zr_hbm, rowsA)
    pltpu.sync_copy(z16_hbm, w16_v)
    _init_shared(rowsA, acc_sh, row0, s)
    _init_shared(w16_v, den_sh, row0, s)
    plsc.subcore_barrier()
    g16 = g16_v[pl.ds(0, 16)]

    def cid(q):
        return jnp.where(q < FULL, wid + q * NW, FULL * NW + wid)

    def fetch_idx(q, src_v, dst_v):
        base = cid(q) * CH
        pltpu.sync_copy(src_hbm.at[pl.ds(base, CH)], src_v)
        pltpu.sync_copy(dst_hbm.at[pl.ds(base, CH)], dst_v)

    def wloop(src_v, dst_v):
        for k in range(CH // 16):
            sv = src_v[pl.ds(k * 16, 16)]
            dv = dst_v[pl.ds(k * 16, 16)]
            hsg = plsc.load_gather(hs_v, [sv])
            hdg = plsc.load_gather(hd_v, [dv])
            mg = g16 + hdg
            mg = jnp.where(mg > 0, mg, mg * 0.2)
            a = hsg + hdg
            a = jnp.where(a > 0, a, a * 0.2)
            w_v[pl.ds(k * 16, 16)] = jnp.exp(a - mg)

    def srow_scatter(rows_v, dst_v):
        for r in range(CH):
            wb = plsc.load_gather(w_v, [jnp.full((16,), r, jnp.int32)])
            w16_v[r, pl.ds(0, 16)] = wb
            for k in range(D // 16):
                rows_v[r, pl.ds(k * 16, 16)] = (
                    rows_v[r, pl.ds(k * 16, 16)] * wb)
        pltpu.async_copy(rows_v, acc_sh.at[dst_v], sem_s, add=True)
        pltpu.sync_copy(w16_v, den_sh.at[dst_v], add=True)

    fetch_idx(0, srcA, dstA)
    pltpu.async_copy(h_hbm.at[srcA], rowsA, sem)

    def half(q, src_v, dst_v, rows_v, nsrc_v, ndst_v, nrows_v):
        @pl.when(q < nchunk)
        def _():
            wloop(src_v, dst_v)
            pltpu.make_async_copy(h_hbm.at[src_v], rows_v, sem).wait()

            @pl.when(q + 1 < nchunk)
            def _():
                @pl.when(q >= 1)
                def _():
                    pltpu.make_async_copy(
                        nrows_v, acc_sh.at[ndst_v], sem_s).wait()

                fetch_idx(q + 1, nsrc_v, ndst_v)
                pltpu.async_copy(h_hbm.at[nsrc_v], nrows_v, sem)

            srow_scatter(rows_v, dst_v)

    def pair(p, _):
        half(2 * p, srcA, dstA, rowsA, srcB, dstB, rowsB)
        half(2 * p + 1, srcB, dstB, rowsB, srcA, dstA, rowsA)
        return 0

    lax.fori_loop(0, (FULL + 2) // 2, pair, 0)

    pltpu.make_async_copy(rowsA, acc_sh.at[dstA], sem_s).wait()
    pltpu.make_async_copy(rowsB, acc_sh.at[dstB], sem_s).wait()
    plsc.subcore_barrier()
    _copy_out(acc_sh, acc0_o, acc1_o, c, row0, s)
    _copy_out(den_sh, den0_o, den1_o, c, row0, s)


_gat_edge_k = pl.kernel(
    _gat_edge,
    out_type=[jax.ShapeDtypeStruct((N, D), jnp.float32),
              jax.ShapeDtypeStruct((N, D), jnp.float32),
              jax.ShapeDtypeStruct((N, DEN_W), jnp.float32),
              jax.ShapeDtypeStruct((N, DEN_W), jnp.float32)],
    mesh=_sc_mesh,
    compiler_params=pltpu.CompilerParams(needs_layout_passes=False, use_tc_tiling_on_sc=False),
    scratch_types=[
        pltpu.VMEM((N,), jnp.float32),
        pltpu.VMEM((N,), jnp.float32),
        pltpu.VMEM((16,), jnp.float32),
        pltpu.VMEM((CH,), jnp.int32),
        pltpu.VMEM((CH,), jnp.int32),
        pltpu.VMEM((CH,), jnp.int32),
        pltpu.VMEM((CH,), jnp.int32),
        pltpu.VMEM((CH,), jnp.float32),
        pltpu.VMEM((CH, D), jnp.float32),
        pltpu.VMEM((CH, D), jnp.float32),
        pltpu.VMEM((CH, DEN_W), jnp.float32),
        pltpu.VMEM_SHARED((N, D), jnp.float32),
        pltpu.VMEM_SHARED((N, DEN_W), jnp.float32),
        pltpu.SemaphoreType.DMA,
        pltpu.SemaphoreType.DMA,
    ])


def _sage_edge(with_cnt, *refs):
    if with_cnt:
        (x_hbm, zr_hbm, z16_hbm, ones_hbm, src_hbm, dst_hbm,
         sacc0_o, sacc1_o, cnt0_o, cnt1_o,
         srcA, dstA, srcB, dstB, rowsA, rowsB, ones_v,
         acc_sh, cnt_sh, sem, sem_s) = refs
    else:
        (x_hbm, zr_hbm, src_hbm, dst_hbm,
         sacc0_o, sacc1_o,
         srcA, dstA, srcB, dstB, rowsA, rowsB,
         acc_sh, sem, sem_s) = refs
    c = lax.axis_index("c")
    s = lax.axis_index("s")
    wid = s * NC + c
    row0 = s * RPT
    nchunk = FULL + jnp.where(wid < EXTRA, 1, 0)

    pltpu.sync_copy(zr_hbm, rowsA)
    _init_shared(rowsA, acc_sh, row0, s)
    if with_cnt:
        pltpu.sync_copy(z16_hbm, ones_v)
        _init_shared(ones_v, cnt_sh, row0, s)
        pltpu.sync_copy(ones_hbm, ones_v)
    plsc.subcore_barrier()

    def cid(q):
        return jnp.where(q < FULL, wid + q * NW, FULL * NW + wid)

    def fetch_idx(q, src_v, dst_v):
        base = cid(q) * CH
        pltpu.sync_copy(src_hbm.at[pl.ds(base, CH)], src_v)
        pltpu.sync_copy(dst_hbm.at[pl.ds(base, CH)], dst_v)

    fetch_idx(0, srcA, dstA)
    pltpu.async_copy(x_hbm.at[srcA], rowsA, sem)

    def half(q, src_v, dst_v, rows_v, nsrc_v, ndst_v, nrows_v):
        @pl.when(q < nchunk)
        def _():
            pltpu.make_async_copy(x_hbm.at[src_v], rows_v, sem).wait()

            @pl.when(q + 1 < nchunk)
            def _():
                @pl.when(q >= 1)
                def _():
                    pltpu.make_async_copy(
                        nrows_v, acc_sh.at[ndst_v], sem_s).wait()

                fetch_idx(q + 1, nsrc_v, ndst_v)
                pltpu.async_copy(x_hbm.at[nsrc_v], nrows_v, sem)

            pltpu.async_copy(rows_v, acc_sh.at[dst_v], sem_s, add=True)
            if with_cnt:
                pltpu.sync_copy(ones_v, cnt_sh.at[dst_v], add=True)

    def pair(p, _):
        half(2 * p, srcA, dstA, rowsA, srcB, dstB, rowsB)
        half(2 * p + 1, srcB, dstB, rowsB, srcA, dstA, rowsA)
        return 0

    lax.fori_loop(0, (FULL + 2) // 2, pair, 0)

    pltpu.make_async_copy(rowsA, acc_sh.at[dstA], sem_s).wait()
    pltpu.make_async_copy(rowsB, acc_sh.at[dstB], sem_s).wait()
    plsc.subcore_barrier()
    _copy_out(acc_sh, sacc0_o, sacc1_o, c, row0, s)
    if with_cnt:
        _copy_out(cnt_sh, cnt0_o, cnt1_o, c, row0, s)


_sc_params = pltpu.CompilerParams(needs_layout_passes=False,
                                  use_tc_tiling_on_sc=False)

_sage_cnt_k = pl.kernel(
    functools.partial(_sage_edge, True),
    out_type=[jax.ShapeDtypeStruct((N, D), jnp.float32),
              jax.ShapeDtypeStruct((N, D), jnp.float32),
              jax.ShapeDtypeStruct((N, DEN_W), jnp.float32),
              jax.ShapeDtypeStruct((N, DEN_W), jnp.float32)],
    mesh=_sc_mesh,
    compiler_params=_sc_params,
    scratch_types=[
        pltpu.VMEM((CH,), jnp.int32),
        pltpu.VMEM((CH,), jnp.int32),
        pltpu.VMEM((CH,), jnp.int32),
        pltpu.VMEM((CH,), jnp.int32),
        pltpu.VMEM((CH, D), jnp.float32),
        pltpu.VMEM((CH, D), jnp.float32),
        pltpu.VMEM((CH, DEN_W), jnp.float32),
        pltpu.VMEM_SHARED((N, D), jnp.float32),
        pltpu.VMEM_SHARED((N, DEN_W), jnp.float32),
        pltpu.SemaphoreType.DMA,
        pltpu.SemaphoreType.DMA,
    ])

_sage_nocnt_k = pl.kernel(
    functools.partial(_sage_edge, False),
    out_type=[jax.ShapeDtypeStruct((N, D), jnp.float32),
              jax.ShapeDtypeStruct((N, D), jnp.float32)],
    mesh=_sc_mesh,
    compiler_params=_sc_params,
    scratch_types=[
        pltpu.VMEM((CH,), jnp.int32),
        pltpu.VMEM((CH,), jnp.int32),
        pltpu.VMEM((CH,), jnp.int32),
        pltpu.VMEM((CH,), jnp.int32),
        pltpu.VMEM((CH, D), jnp.float32),
        pltpu.VMEM((CH, D), jnp.float32),
        pltpu.VMEM_SHARED((N, D), jnp.float32),
        pltpu.SemaphoreType.DMA,
        pltpu.SemaphoreType.DMA,
    ])

_BPW = B // NW


def _readout(x4_hbm, xv_hbm, fvm_hbm, it0_hbm, it1_hbm, it2_hbm,
             user_o, pos_o, vid_o,
             fvm_v, i0_v, i1_v, i2_v, vi_v, u_v, p_v, v_v, sem):
    c = lax.axis_index("c")
    s = lax.axis_index("s")
    wid = s * NC + c
    base = wid * _BPW
    pltpu.sync_copy(fvm_hbm, fvm_v)
    pltpu.sync_copy(it0_hbm.at[pl.ds(base, _BPW)], i0_v)
    pltpu.sync_copy(it1_hbm.at[pl.ds(base, _BPW)], i1_v)
    pltpu.sync_copy(it2_hbm.at[pl.ds(base, _BPW)], i2_v)
    for k in range(_BPW // 16):
        iv = i1_v[pl.ds(k * 16, 16)]
        vi_v[pl.ds(k * 16, 16)] = plsc.load_gather(fvm_v, [iv])
    pltpu.async_copy(x4_hbm.at[i0_v], u_v, sem).wait()
    pltpu.async_copy(x4_hbm.at[i2_v], p_v, sem).wait()
    pltpu.async_copy(xv_hbm.at[vi_v], v_v, sem).wait()
    pltpu.sync_copy(u_v, user_o.at[pl.ds(base, _BPW)])
    pltpu.sync_copy(p_v, pos_o.at[pl.ds(base, _BPW)])
    pltpu.sync_copy(v_v, vid_o.at[pl.ds(base, _BPW)])


_readout_k = pl.kernel(
    _readout,
    out_type=[jax.ShapeDtypeStruct((B, D), jnp.float32),
              jax.ShapeDtypeStruct((B, D), jnp.float32),
              jax.ShapeDtypeStruct((B, D), jnp.float32)],
    mesh=_sc_mesh,
    compiler_params=pltpu.CompilerParams(needs_layout_passes=False, use_tc_tiling_on_sc=False),
    scratch_types=[
        pltpu.VMEM((NV,), jnp.int32),
        pltpu.VMEM((_BPW,), jnp.int32),
        pltpu.VMEM((_BPW,), jnp.int32),
        pltpu.VMEM((_BPW,), jnp.int32),
        pltpu.VMEM((_BPW,), jnp.int32),
        pltpu.VMEM((_BPW, D), jnp.float32),
        pltpu.VMEM((_BPW, D), jnp.float32),
        pltpu.VMEM((_BPW, D), jnp.float32),
        pltpu.SemaphoreType.DMA,
    ])


def kernel(item, uh_edge_index, v_uh_edge_index, feature_video_mapping,
           features, u_h_embedding, W_tv, b_tv, W_g1, a1_src, a1_dst,
           W_s2l, W_s2r, W_g3, a3_src, a3_dst, W_s4l, W_s4r,
           W_uv, b_uv, W_uh, b_uh):
    src1 = v_uh_edge_index[0]
    dst1 = v_uh_edge_index[1]
    src2 = uh_edge_index[0]
    dst2 = uh_edge_index[1]
    it0 = item[:, 0].astype(jnp.int32)
    it1 = item[:, 1].astype(jnp.int32)
    it2 = item[:, 2].astype(jnp.int32)

    xv, x0 = _feat_k(features, u_h_embedding, W_tv, b_tv[None, :])

    h1, hs1, hd1, gm1 = _prep_k(x0, W_g1, a1_src[:, None], a1_dst[:, None])
    zr = jnp.zeros((CH, D), jnp.float32)
    z16 = jnp.zeros((CH, DEN_W), jnp.float32)
    ones16 = jnp.ones((CH, DEN_W), jnp.float32)
    ga0, ga1, gd0, gd1 = _gat_edge_k(h1, hs1.reshape(N), hd1.reshape(N),
                                     jnp.broadcast_to(gm1.reshape(1), (16,)),
                                     zr, z16, src1, dst1)
    x1 = _postgat_k(ga0, ga1, gd0, gd1)

    sa0, sa1, sc0, sc1 = _sage_cnt_k(x1, zr, z16, ones16, src2, dst2)
    x2 = _postsage_k(sa0, sa1, sc0, sc1, x1, W_s2l, W_s2r)

    h3, hs3, hd3, gm3 = _prep_k(x2, W_g3, a3_src[:, None], a3_dst[:, None])
    gb0, gb1, ge0, ge1 = _gat_edge_k(h3, hs3.reshape(N), hd3.reshape(N),
                                     jnp.broadcast_to(gm3.reshape(1), (16,)),
                                     zr, z16, src1, dst1)
    x3 = _postgat_k(gb0, gb1, ge0, ge1)

    sb0, sb1 = _sage_nocnt_k(x3, zr, src2, dst2)
    x4 = _postsage_k(sb0, sb1, sc0, sc1, x3, W_s4l, W_s4r)

    user, pos, vid = _readout_k(x4, xv, feature_video_mapping,
                                it0, it1, it2)
    scores = _score_k(user, pos, vid, W_uv, b_uv[None, :],
                      W_uh, b_uh[None, :])
    return scores.reshape(B)


# speedup vs baseline: 9.9702x; 1.1415x over previous
import functools

import jax
import jax.numpy as jnp
from jax import lax
from jax.experimental import pallas as pl
from jax.experimental.pallas import tpu as pltpu
from jax.experimental.pallas import tpu_sc as plsc

NU, NH, NV = 4000, 1000, 5000
NUH = NU + NH
N = NUH + NV
D, DF, E, B = 128, 512, 320000, 1024

NC, NS = 2, 16
NW = NC * NS
CH = 64
NCHUNK = E // CH
FULL = NCHUNK // NW
EXTRA = NCHUNK - FULL * NW
RPT = 624
DEN_W = 16

_HI = lax.Precision.HIGHEST


def _lrelu(x, s):
    return jnp.where(x > 0, x, x * s)


def _feat_body(feat, uh, Wtv, btv, xv_o, x0_o):
    xv = _lrelu(jnp.dot(feat[...], Wtv[...], precision=_HI) + btv[...], 0.01)
    xv_o[...] = xv
    x = jnp.concatenate([uh[...], xv], axis=0)
    nrm = jnp.sqrt(jnp.sum(x * x, axis=1, keepdims=True))
    x0_o[...] = x / jnp.maximum(nrm, 1e-12)


_feat_k = pl.pallas_call(
    _feat_body,
    out_shape=[jax.ShapeDtypeStruct((NV, D), jnp.float32),
               jax.ShapeDtypeStruct((N, D), jnp.float32)])


def _prep_body(x, Wg, a_s, a_d, h_o, hs_o, hd_o, gm_o):
    i = pl.program_id(0)
    h = jnp.dot(x[...], Wg[...], precision=_HI)
    h_o[...] = h
    hs = jnp.dot(h, a_s[...], precision=_HI)
    hd = jnp.dot(h, a_d[...], precision=_HI)
    hs_o[...] = hs
    hd_o[...] = hd

    @pl.when(i == 0)
    def _():
        gm_o[...] = jnp.full((1, 1), -jnp.inf)

    gm_o[...] = jnp.maximum(gm_o[...], jnp.max(hs))


_RB = 2000

_prep_k = pl.pallas_call(
    _prep_body,
    grid=(N // _RB,),
    in_specs=[pl.BlockSpec((_RB, D), lambda i: (i, 0)),
              pl.BlockSpec((D, D), lambda i: (0, 0)),
              pl.BlockSpec((D, 1), lambda i: (0, 0)),
              pl.BlockSpec((D, 1), lambda i: (0, 0))],
    out_specs=[pl.BlockSpec((_RB, D), lambda i: (i, 0)),
               pl.BlockSpec((_RB, 1), lambda i: (i, 0)),
               pl.BlockSpec((_RB, 1), lambda i: (i, 0)),
               pl.BlockSpec((1, 1), lambda i: (0, 0))],
    out_shape=[jax.ShapeDtypeStruct((N, D), jnp.float32),
               jax.ShapeDtypeStruct((N, 1), jnp.float32),
               jax.ShapeDtypeStruct((N, 1), jnp.float32),
               jax.ShapeDtypeStruct((1, 1), jnp.float32)])


def _postgat_body(acc0, acc1, den0, den1, x_o):
    a = acc0[...] + acc1[...]
    d = den0[:, 0] + den1[:, 0]
    x_o[...] = _lrelu(a / (d[:, None] + 1e-16), 0.01)


_postgat_k = pl.pallas_call(
    _postgat_body,
    grid=(N // _RB,),
    in_specs=[pl.BlockSpec((_RB, D), lambda i: (i, 0)),
              pl.BlockSpec((_RB, D), lambda i: (i, 0)),
              pl.BlockSpec((_RB, DEN_W), lambda i: (i, 0)),
              pl.BlockSpec((_RB, DEN_W), lambda i: (i, 0))],
    out_specs=pl.BlockSpec((_RB, D), lambda i: (i, 0)),
    out_shape=jax.ShapeDtypeStruct((N, D), jnp.float32))


def _postsage_body(sacc0, sacc1, cnt0, cnt1, x, Wl, Wr, x_o):
    s = sacc0[...] + sacc1[...]
    c = cnt0[:, 0] + cnt1[:, 0]
    mean = s / jnp.maximum(c, 1.0)[:, None]
    x_o[...] = _lrelu(jnp.dot(mean, Wl[...], precision=_HI)
                      + jnp.dot(x[...], Wr[...], precision=_HI), 0.01)


_postsage_k = pl.pallas_call(
    _postsage_body,
    grid=(N // _RB,),
    in_specs=[pl.BlockSpec((_RB, D), lambda i: (i, 0)),
              pl.BlockSpec((_RB, D), lambda i: (i, 0)),
              pl.BlockSpec((_RB, DEN_W), lambda i: (i, 0)),
              pl.BlockSpec((_RB, DEN_W), lambda i: (i, 0)),
              pl.BlockSpec((_RB, D), lambda i: (i, 0)),
              pl.BlockSpec((D, D), lambda i: (0, 0)),
              pl.BlockSpec((D, D), lambda i: (0, 0))],
    out_specs=pl.BlockSpec((_RB, D), lambda i: (i, 0)),
    out_shape=jax.ShapeDtypeStruct((N, D), jnp.float32))


def _score_body(user, pos, vid, Wuv, buv, Wuh, buh, out_o):
    usv = _lrelu(jnp.dot(jnp.concatenate([vid[...], user[...]], axis=1),
                         Wuv[...], precision=_HI) + buv[...], 0.01)
    usp = _lrelu(jnp.dot(jnp.concatenate([pos[...], user[...]], axis=1),
                         Wuh[...], precision=_HI) + buh[...], 0.01)
    out_o[...] = jnp.sum(usv * usp, axis=1, keepdims=True)


_score_k = pl.pallas_call(
    _score_body,
    out_shape=jax.ShapeDtypeStruct((B, 1), jnp.float32))


_sc_mesh = plsc.VectorSubcoreMesh(core_axis_name="c", subcore_axis_name="s")
_Z16F = functools.partial(jnp.zeros, (16,), jnp.float32)


def _init_shared(src_ref, sh_ref, row0, s):
    for j in range(RPT // CH):
        pltpu.sync_copy(src_ref, sh_ref.at[pl.ds(row0 + j * CH, CH)])
    rem = RPT - (RPT // CH) * CH
    if rem:
        pltpu.sync_copy(src_ref.at[pl.ds(0, rem)],
                        sh_ref.at[pl.ds(row0 + (RPT // CH) * CH, rem)])

    @pl.when(s == NS - 1)
    def _():
        pltpu.sync_copy(src_ref.at[pl.ds(0, N - NS * RPT)],
                        sh_ref.at[pl.ds(NS * RPT, N - NS * RPT)])


def _copy_out(sh_ref, out0_ref, out1_ref, c, row0, s):
    def emit(out_ref):
        pltpu.sync_copy(sh_ref.at[pl.ds(row0, RPT)],
                        out_ref.at[pl.ds(row0, RPT)])

        @pl.when(s == NS - 1)
        def _():
            pltpu.sync_copy(sh_ref.at[pl.ds(NS * RPT, N - NS * RPT)],
                            out_ref.at[pl.ds(NS * RPT, N - NS * RPT)])

    @pl.when(c == 0)
    def _():
        emit(out0_ref)

    @pl.when(c == 1)
    def _():
        emit(out1_ref)


def _gat_edge(h_hbm, hs_hbm, hd_hbm, g16_hbm, zr_hbm, z16_hbm, sd_hbm,
              acc0_o, acc1_o, den0_o, den1_o,
              hs_v, hd_v, g16_v, sdA, sdB, w_v,
              rowsA, rowsB, w16_v, acc_sh, den_sh, sem, sem_s):
    c = lax.axis_index("c")
    s = lax.axis_index("s")
    wid = s * NC + c
    row0 = s * RPT
    nchunk = FULL + jnp.where(wid < EXTRA, 1, 0)

    pltpu.sync_copy(hs_hbm, hs_v)
    pltpu.sync_copy(hd_hbm, hd_v)
    pltpu.sync_copy(g16_hbm, g16_v)
    pltpu.sync_copy(zr_hbm, rowsA)
    pltpu.sync_copy(z16_hbm, w16_v)
    _init_shared(rowsA, acc_sh, row0, s)
    _init_shared(w16_v, den_sh, row0, s)
    plsc.subcore_barrier()
    g16 = g16_v[pl.ds(0, 16)]

    def cid(q):
        return jnp.where(q < FULL, wid + q * NW, FULL * NW + wid)

    def fetch_idx(q, sd_v):
        pltpu.sync_copy(sd_hbm.at[pl.ds(cid(q), 1)], sd_v)

    def wloop(sd_v):
        for k in range(CH // 16):
            sv = sd_v[0, 0, pl.ds(k * 16, 16)]
            dv = sd_v[0, 1, pl.ds(k * 16, 16)]
            hsg = plsc.load_gather(hs_v, [sv])
            hdg = plsc.load_gather(hd_v, [dv])
            mg = g16 + hdg
            mg = jnp.where(mg > 0, mg, mg * 0.2)
            a = hsg + hdg
            a = jnp.where(a > 0, a, a * 0.2)
            w_v[pl.ds(k * 16, 16)] = jnp.exp(a - mg)

    def srow_scatter(rows_v, sd_v):
        for r in range(CH):
            wb = plsc.load_gather(w_v, [jnp.full((16,), r, jnp.int32)])
            w16_v[r, pl.ds(0, 16)] = wb
            for k in range(D // 16):
                rows_v[r, pl.ds(k * 16, 16)] = (
                    rows_v[r, pl.ds(k * 16, 16)] * wb)
        pltpu.async_copy(rows_v, acc_sh.at[sd_v.at[0, 1]], sem_s, add=True)
        pltpu.sync_copy(w16_v, den_sh.at[sd_v.at[0, 1]], add=True)

    fetch_idx(0, sdA)
    pltpu.async_copy(h_hbm.at[sdA.at[0, 0]], rowsA, sem)

    def half(q, sd_v, rows_v, nsd_v, nrows_v):
        @pl.when(q < nchunk)
        def _():
            wloop(sd_v)
            pltpu.make_async_copy(h_hbm.at[sd_v.at[0, 0]], rows_v, sem).wait()

            @pl.when(q + 1 < nchunk)
            def _():
                @pl.when(q >= 1)
                def _():
                    pltpu.make_async_copy(
                        nrows_v, acc_sh.at[nsd_v.at[0, 1]], sem_s).wait()

                fetch_idx(q + 1, nsd_v)
                pltpu.async_copy(h_hbm.at[nsd_v.at[0, 0]], nrows_v, sem)

            srow_scatter(rows_v, sd_v)

    def pair(p, _):
        half(2 * p, sdA, rowsA, sdB, rowsB)
        half(2 * p + 1, sdB, rowsB, sdA, rowsA)
        return 0

    lax.fori_loop(0, (FULL + 2) // 2, pair, 0)

    pltpu.make_async_copy(rowsA, acc_sh.at[sdA.at[0, 1]], sem_s).wait()
    pltpu.make_async_copy(rowsB, acc_sh.at[sdB.at[0, 1]], sem_s).wait()
    plsc.subcore_barrier()
    _copy_out(acc_sh, acc0_o, acc1_o, c, row0, s)
    _copy_out(den_sh, den0_o, den1_o, c, row0, s)


_gat_edge_k = pl.kernel(
    _gat_edge,
    out_type=[jax.ShapeDtypeStruct((N, D), jnp.float32),
              jax.ShapeDtypeStruct((N, D), jnp.float32),
              jax.ShapeDtypeStruct((N, DEN_W), jnp.float32),
              jax.ShapeDtypeStruct((N, DEN_W), jnp.float32)],
    mesh=_sc_mesh,
    compiler_params=pltpu.CompilerParams(needs_layout_passes=False, use_tc_tiling_on_sc=False),
    scratch_types=[
        pltpu.VMEM((N,), jnp.float32),
        pltpu.VMEM((N,), jnp.float32),
        pltpu.VMEM((16,), jnp.float32),
        pltpu.VMEM((1, 2, CH), jnp.int32),
        pltpu.VMEM((1, 2, CH), jnp.int32),
        pltpu.VMEM((CH,), jnp.float32),
        pltpu.VMEM((CH, D), jnp.float32),
        pltpu.VMEM((CH, D), jnp.float32),
        pltpu.VMEM((CH, DEN_W), jnp.float32),
        pltpu.VMEM_SHARED((N, D), jnp.float32),
        pltpu.VMEM_SHARED((N, DEN_W), jnp.float32),
        pltpu.SemaphoreType.DMA,
        pltpu.SemaphoreType.DMA,
    ])


def _sage_edge(with_cnt, *refs):
    if with_cnt:
        (x_hbm, zr_hbm, z16_hbm, ones_hbm, sd_hbm,
         sacc0_o, sacc1_o, cnt0_o, cnt1_o,
         sdA, sdB, rowsA, rowsB, ones_v,
         acc_sh, cnt_sh, sem, sem_s) = refs
    else:
        (x_hbm, zr_hbm, sd_hbm,
         sacc0_o, sacc1_o,
         sdA, sdB, rowsA, rowsB,
         acc_sh, sem, sem_s) = refs
    c = lax.axis_index("c")
    s = lax.axis_index("s")
    wid = s * NC + c
    row0 = s * RPT
    nchunk = FULL + jnp.where(wid < EXTRA, 1, 0)

    pltpu.sync_copy(zr_hbm, rowsA)
    _init_shared(rowsA, acc_sh, row0, s)
    if with_cnt:
        pltpu.sync_copy(z16_hbm, ones_v)
        _init_shared(ones_v, cnt_sh, row0, s)
        pltpu.sync_copy(ones_hbm, ones_v)
    plsc.subcore_barrier()

    def cid(q):
        return jnp.where(q < FULL, wid + q * NW, FULL * NW + wid)

    def fetch_idx(q, sd_v):
        pltpu.sync_copy(sd_hbm.at[pl.ds(cid(q), 1)], sd_v)

    fetch_idx(0, sdA)
    pltpu.async_copy(x_hbm.at[sdA.at[0, 0]], rowsA, sem)

    def half(q, sd_v, rows_v, nsd_v, nrows_v):
        @pl.when(q < nchunk)
        def _():
            pltpu.make_async_copy(x_hbm.at[sd_v.at[0, 0]], rows_v, sem).wait()

            @pl.when(q + 1 < nchunk)
            def _():
                @pl.when(q >= 1)
                def _():
                    pltpu.make_async_copy(
                        nrows_v, acc_sh.at[nsd_v.at[0, 1]], sem_s).wait()

                fetch_idx(q + 1, nsd_v)
                pltpu.async_copy(x_hbm.at[nsd_v.at[0, 0]], nrows_v, sem)

            pltpu.async_copy(rows_v, acc_sh.at[sd_v.at[0, 1]], sem_s, add=True)
            if with_cnt:
                pltpu.sync_copy(ones_v, cnt_sh.at[sd_v.at[0, 1]], add=True)

    def pair(p, _):
        half(2 * p, sdA, rowsA, sdB, rowsB)
        half(2 * p + 1, sdB, rowsB, sdA, rowsA)
        return 0

    lax.fori_loop(0, (FULL + 2) // 2, pair, 0)

    pltpu.make_async_copy(rowsA, acc_sh.at[sdA.at[0, 1]], sem_s).wait()
    pltpu.make_async_copy(rowsB, acc_sh.at[sdB.at[0, 1]], sem_s).wait()
    plsc.subcore_barrier()
    _copy_out(acc_sh, sacc0_o, sacc1_o, c, row0, s)
    if with_cnt:
        _copy_out(cnt_sh, cnt0_o, cnt1_o, c, row0, s)


_sc_params = pltpu.CompilerParams(needs_layout_passes=False,
                                  use_tc_tiling_on_sc=False)

_sage_cnt_k = pl.kernel(
    functools.partial(_sage_edge, True),
    out_type=[jax.ShapeDtypeStruct((N, D), jnp.float32),
              jax.ShapeDtypeStruct((N, D), jnp.float32),
              jax.ShapeDtypeStruct((N, DEN_W), jnp.float32),
              jax.ShapeDtypeStruct((N, DEN_W), jnp.float32)],
    mesh=_sc_mesh,
    compiler_params=_sc_params,
    scratch_types=[
        pltpu.VMEM((1, 2, CH), jnp.int32),
        pltpu.VMEM((1, 2, CH), jnp.int32),
        pltpu.VMEM((CH, D), jnp.float32),
        pltpu.VMEM((CH, D), jnp.float32),
        pltpu.VMEM((CH, DEN_W), jnp.float32),
        pltpu.VMEM_SHARED((N, D), jnp.float32),
        pltpu.VMEM_SHARED((N, DEN_W), jnp.float32),
        pltpu.SemaphoreType.DMA,
        pltpu.SemaphoreType.DMA,
    ])

_sage_nocnt_k = pl.kernel(
    functools.partial(_sage_edge, False),
    out_type=[jax.ShapeDtypeStruct((N, D), jnp.float32),
              jax.ShapeDtypeStruct((N, D), jnp.float32)],
    mesh=_sc_mesh,
    compiler_params=_sc_params,
    scratch_types=[
        pltpu.VMEM((1, 2, CH), jnp.int32),
        pltpu.VMEM((1, 2, CH), jnp.int32),
        pltpu.VMEM((CH, D), jnp.float32),
        pltpu.VMEM((CH, D), jnp.float32),
        pltpu.VMEM_SHARED((N, D), jnp.float32),
        pltpu.SemaphoreType.DMA,
        pltpu.SemaphoreType.DMA,
    ])

_BPW = B // NW


def _readout(x4_hbm, xv_hbm, fvm_hbm, it0_hbm, it1_hbm, it2_hbm,
             user_o, pos_o, vid_o,
             fvm_v, i0_v, i1_v, i2_v, vi_v, u_v, p_v, v_v, sem):
    c = lax.axis_index("c")
    s = lax.axis_index("s")
    wid = s * NC + c
    base = wid * _BPW
    pltpu.sync_copy(fvm_hbm, fvm_v)
    pltpu.sync_copy(it0_hbm.at[pl.ds(base, _BPW)], i0_v)
    pltpu.sync_copy(it1_hbm.at[pl.ds(base, _BPW)], i1_v)
    pltpu.sync_copy(it2_hbm.at[pl.ds(base, _BPW)], i2_v)
    for k in range(_BPW // 16):
        iv = i1_v[pl.ds(k * 16, 16)]
        vi_v[pl.ds(k * 16, 16)] = plsc.load_gather(fvm_v, [iv])
    pltpu.async_copy(x4_hbm.at[i0_v], u_v, sem).wait()
    pltpu.async_copy(x4_hbm.at[i2_v], p_v, sem).wait()
    pltpu.async_copy(xv_hbm.at[vi_v], v_v, sem).wait()
    pltpu.sync_copy(u_v, user_o.at[pl.ds(base, _BPW)])
    pltpu.sync_copy(p_v, pos_o.at[pl.ds(base, _BPW)])
    pltpu.sync_copy(v_v, vid_o.at[pl.ds(base, _BPW)])


_readout_k = pl.kernel(
    _readout,
    out_type=[jax.ShapeDtypeStruct((B, D), jnp.float32),
              jax.ShapeDtypeStruct((B, D), jnp.float32),
              jax.ShapeDtypeStruct((B, D), jnp.float32)],
    mesh=_sc_mesh,
    compiler_params=pltpu.CompilerParams(needs_layout_passes=False, use_tc_tiling_on_sc=False),
    scratch_types=[
        pltpu.VMEM((NV,), jnp.int32),
        pltpu.VMEM((_BPW,), jnp.int32),
        pltpu.VMEM((_BPW,), jnp.int32),
        pltpu.VMEM((_BPW,), jnp.int32),
        pltpu.VMEM((_BPW,), jnp.int32),
        pltpu.VMEM((_BPW, D), jnp.float32),
        pltpu.VMEM((_BPW, D), jnp.float32),
        pltpu.VMEM((_BPW, D), jnp.float32),
        pltpu.SemaphoreType.DMA,
    ])


def kernel(item, uh_edge_index, v_uh_edge_index, feature_video_mapping,
           features, u_h_embedding, W_tv, b_tv, W_g1, a1_src, a1_dst,
           W_s2l, W_s2r, W_g3, a3_src, a3_dst, W_s4l, W_s4r,
           W_uv, b_uv, W_uh, b_uh):
    sd1 = jnp.stack([v_uh_edge_index[0].reshape(NCHUNK, CH),
                     v_uh_edge_index[1].reshape(NCHUNK, CH)], axis=1)
    sd2 = jnp.stack([uh_edge_index[0].reshape(NCHUNK, CH),
                     uh_edge_index[1].reshape(NCHUNK, CH)], axis=1)
    it0 = item[:, 0].astype(jnp.int32)
    it1 = item[:, 1].astype(jnp.int32)
    it2 = item[:, 2].astype(jnp.int32)

    xv, x0 = _feat_k(features, u_h_embedding, W_tv, b_tv[None, :])

    h1, hs1, hd1, gm1 = _prep_k(x0, W_g1, a1_src[:, None], a1_dst[:, None])
    zr = jnp.zeros((CH, D), jnp.float32)
    z16 = jnp.zeros((CH, DEN_W), jnp.float32)
    ones16 = jnp.ones((CH, DEN_W), jnp.float32)
    ga0, ga1, gd0, gd1 = _gat_edge_k(h1, hs1.reshape(N), hd1.reshape(N),
                                     jnp.broadcast_to(gm1.reshape(1), (16,)),
                                     zr, z16, sd1)
    x1 = _postgat_k(ga0, ga1, gd0, gd1)

    sa0, sa1, sc0, sc1 = _sage_cnt_k(x1, zr, z16, ones16, sd2)
    x2 = _postsage_k(sa0, sa1, sc0, sc1, x1, W_s2l, W_s2r)

    h3, hs3, hd3, gm3 = _prep_k(x2, W_g3, a3_src[:, None], a3_dst[:, None])
    gb0, gb1, ge0, ge1 = _gat_edge_k(h3, hs3.reshape(N), hd3.reshape(N),
                                     jnp.broadcast_to(gm3.reshape(1), (16,)),
                                     zr, z16, sd1)
    x3 = _postgat_k(gb0, gb1, ge0, ge1)

    sb0, sb1 = _sage_nocnt_k(x3, zr, sd2)
    x4 = _postsage_k(sb0, sb1, sc0, sc1, x3, W_s4l, W_s4r)

    user, pos, vid = _readout_k(x4, xv, feature_video_mapping,
                                it0, it1, it2)
    scores = _score_k(user, pos, vid, W_uv, b_uv[None, :],
                      W_uh, b_uh[None, :])
    return scores.reshape(B)


# speedup vs baseline: 10.9783x; 1.1011x over previous
import functools

import jax
import jax.numpy as jnp
from jax import lax
from jax.experimental import pallas as pl
from jax.experimental.pallas import tpu as pltpu
from jax.experimental.pallas import tpu_sc as plsc

NU, NH, NV = 4000, 1000, 5000
NUH = NU + NH
N = NUH + NV
D, DF, E, B = 128, 512, 320000, 1024

NC, NS = 2, 16
NW = NC * NS
CH = 64
NCHUNK = E // CH
FULL = NCHUNK // NW
EXTRA = NCHUNK - FULL * NW
RPT = 624
DEN_W = 16

_HI = lax.Precision.HIGHEST


def _lrelu(x, s):
    return jnp.where(x > 0, x, x * s)


def _feat_body(feat, uh, Wtv, btv, xv_o, x0_o):
    xv = _lrelu(jnp.dot(feat[...], Wtv[...], precision=_HI) + btv[...], 0.01)
    xv_o[...] = xv
    x = jnp.concatenate([uh[...], xv], axis=0)
    nrm = jnp.sqrt(jnp.sum(x * x, axis=1, keepdims=True))
    x0_o[...] = x / jnp.maximum(nrm, 1e-12)


_feat_k = pl.pallas_call(
    _feat_body,
    out_shape=[jax.ShapeDtypeStruct((NV, D), jnp.float32),
               jax.ShapeDtypeStruct((N, D), jnp.float32)])


def _prep_body(x, Wg, a_s, a_d, h_o, hs_o, hd_o, gm_o):
    i = pl.program_id(0)
    h = jnp.dot(x[...], Wg[...], precision=_HI)
    h_o[...] = h
    hs = jnp.dot(h, a_s[...], precision=_HI)
    hd = jnp.dot(h, a_d[...], precision=_HI)
    hs_o[...] = hs
    hd_o[...] = hd

    @pl.when(i == 0)
    def _():
        gm_o[...] = jnp.full((1, 1), -jnp.inf)

    gm_o[...] = jnp.maximum(gm_o[...], jnp.max(hs))


_RB = 2000

_prep_k = pl.pallas_call(
    _prep_body,
    grid=(N // _RB,),
    in_specs=[pl.BlockSpec((_RB, D), lambda i: (i, 0)),
              pl.BlockSpec((D, D), lambda i: (0, 0)),
              pl.BlockSpec((D, 1), lambda i: (0, 0)),
              pl.BlockSpec((D, 1), lambda i: (0, 0))],
    out_specs=[pl.BlockSpec((_RB, D), lambda i: (i, 0)),
               pl.BlockSpec((_RB, 1), lambda i: (i, 0)),
               pl.BlockSpec((_RB, 1), lambda i: (i, 0)),
               pl.BlockSpec((1, 1), lambda i: (0, 0))],
    out_shape=[jax.ShapeDtypeStruct((N, D), jnp.float32),
               jax.ShapeDtypeStruct((N, 1), jnp.float32),
               jax.ShapeDtypeStruct((N, 1), jnp.float32),
               jax.ShapeDtypeStruct((1, 1), jnp.float32)])


def _postgat_body(acc0, acc1, den0, den1, x_o):
    a = acc0[...] + acc1[...]
    d = den0[:, 0] + den1[:, 0]
    x_o[...] = _lrelu(a / (d[:, None] + 1e-16), 0.01)


_postgat_k = pl.pallas_call(
    _postgat_body,
    grid=(N // _RB,),
    in_specs=[pl.BlockSpec((_RB, D), lambda i: (i, 0)),
              pl.BlockSpec((_RB, D), lambda i: (i, 0)),
              pl.BlockSpec((_RB, DEN_W), lambda i: (i, 0)),
              pl.BlockSpec((_RB, DEN_W), lambda i: (i, 0))],
    out_specs=pl.BlockSpec((_RB, D), lambda i: (i, 0)),
    out_shape=jax.ShapeDtypeStruct((N, D), jnp.float32))


def _postsage_body(sacc0, sacc1, cnt0, cnt1, x, Wl, Wr, x_o):
    s = sacc0[...] + sacc1[...]
    c = cnt0[:, 0] + cnt1[:, 0]
    mean = s / jnp.maximum(c, 1.0)[:, None]
    x_o[...] = _lrelu(jnp.dot(mean, Wl[...], precision=_HI)
                      + jnp.dot(x[...], Wr[...], precision=_HI), 0.01)


_postsage_k = pl.pallas_call(
    _postsage_body,
    grid=(N // _RB,),
    in_specs=[pl.BlockSpec((_RB, D), lambda i: (i, 0)),
              pl.BlockSpec((_RB, D), lambda i: (i, 0)),
              pl.BlockSpec((_RB, DEN_W), lambda i: (i, 0)),
              pl.BlockSpec((_RB, DEN_W), lambda i: (i, 0)),
              pl.BlockSpec((_RB, D), lambda i: (i, 0)),
              pl.BlockSpec((D, D), lambda i: (0, 0)),
              pl.BlockSpec((D, D), lambda i: (0, 0))],
    out_specs=pl.BlockSpec((_RB, D), lambda i: (i, 0)),
    out_shape=jax.ShapeDtypeStruct((N, D), jnp.float32))


def _score_body(user, pos, vid, Wuv, buv, Wuh, buh, out_o):
    usv = _lrelu(jnp.dot(jnp.concatenate([vid[...], user[...]], axis=1),
                         Wuv[...], precision=_HI) + buv[...], 0.01)
    usp = _lrelu(jnp.dot(jnp.concatenate([pos[...], user[...]], axis=1),
                         Wuh[...], precision=_HI) + buh[...], 0.01)
    out_o[...] = jnp.sum(usv * usp, axis=1, keepdims=True)


_score_k = pl.pallas_call(
    _score_body,
    out_shape=jax.ShapeDtypeStruct((B, 1), jnp.float32))


_sc_mesh = plsc.VectorSubcoreMesh(core_axis_name="c", subcore_axis_name="s")
_Z16F = functools.partial(jnp.zeros, (16,), jnp.float32)


def _init_shared(src_ref, sh_ref, row0, s):
    for j in range(RPT // CH):
        pltpu.sync_copy(src_ref, sh_ref.at[pl.ds(row0 + j * CH, CH)])
    rem = RPT - (RPT // CH) * CH
    if rem:
        pltpu.sync_copy(src_ref.at[pl.ds(0, rem)],
                        sh_ref.at[pl.ds(row0 + (RPT // CH) * CH, rem)])

    @pl.when(s == NS - 1)
    def _():
        pltpu.sync_copy(src_ref.at[pl.ds(0, N - NS * RPT)],
                        sh_ref.at[pl.ds(NS * RPT, N - NS * RPT)])


def _copy_out(sh_ref, out0_ref, out1_ref, c, row0, s):
    def emit(out_ref):
        pltpu.sync_copy(sh_ref.at[pl.ds(row0, RPT)],
                        out_ref.at[pl.ds(row0, RPT)])

        @pl.when(s == NS - 1)
        def _():
            pltpu.sync_copy(sh_ref.at[pl.ds(NS * RPT, N - NS * RPT)],
                            out_ref.at[pl.ds(NS * RPT, N - NS * RPT)])

    @pl.when(c == 0)
    def _():
        emit(out0_ref)

    @pl.when(c == 1)
    def _():
        emit(out1_ref)


def _gat_edge(h_hbm, hs_hbm, hd_hbm, g16_hbm, zr_hbm, z16_hbm, sd_hbm,
              acc0_o, acc1_o, den0_o, den1_o,
              hs_v, hd_v, g16_v, sdA, sdB, w_v,
              rowsA, rowsB, w16_v, acc_sh, den_sh, sem, sem_s):
    c = lax.axis_index("c")
    s = lax.axis_index("s")
    wid = s * NC + c
    row0 = s * RPT
    nchunk = FULL + jnp.where(wid < EXTRA, 1, 0)

    pltpu.sync_copy(hs_hbm, hs_v)
    pltpu.sync_copy(hd_hbm, hd_v)
    pltpu.sync_copy(g16_hbm, g16_v)
    pltpu.sync_copy(zr_hbm, rowsA)
    pltpu.sync_copy(z16_hbm, w16_v)
    _init_shared(rowsA, acc_sh, row0, s)
    _init_shared(w16_v, den_sh, row0, s)
    plsc.subcore_barrier()
    g16 = g16_v[pl.ds(0, 16)]

    def cid(q):
        return jnp.where(q < FULL, wid + q * NW, FULL * NW + wid)

    def fetch_idx(q, sd_v):
        pltpu.sync_copy(sd_hbm.at[pl.ds(cid(q), 1)], sd_v)

    def wloop(sd_v):
        for k in range(CH // 16):
            sv = sd_v[0, 0, pl.ds(k * 16, 16)]
            dv = sd_v[0, 1, pl.ds(k * 16, 16)]
            hsg = plsc.load_gather(hs_v, [sv])
            hdg = plsc.load_gather(hd_v, [dv])
            mg = g16 + hdg
            mg = jnp.where(mg > 0, mg, mg * 0.2)
            a = hsg + hdg
            a = jnp.where(a > 0, a, a * 0.2)
            w_v[pl.ds(k * 16, 16)] = jnp.exp(a - mg)

    def srow_scatter(rows_v, sd_v):
        for r in range(CH):
            wb = plsc.load_gather(w_v, [jnp.full((16,), r, jnp.int32)])
            w16_v[r, pl.ds(0, 16)] = wb
            for k in range(D // 16):
                rows_v[r, pl.ds(k * 16, 16)] = (
                    rows_v[r, pl.ds(k * 16, 16)] * wb)
        pltpu.async_copy(rows_v, acc_sh.at[sd_v.at[0, 1]], sem_s, add=True)
        pltpu.sync_copy(w16_v, den_sh.at[sd_v.at[0, 1]], add=True)

    fetch_idx(0, sdA)
    pltpu.async_copy(h_hbm.at[sdA.at[0, 0]], rowsA, sem)

    def half(q, sd_v, rows_v, nsd_v, nrows_v):
        @pl.when(q < nchunk)
        def _():
            wloop(sd_v)
            pltpu.make_async_copy(h_hbm.at[sd_v.at[0, 0]], rows_v, sem).wait()

            @pl.when(q + 1 < nchunk)
            def _():
                @pl.when(q >= 1)
                def _():
                    pltpu.make_async_copy(
                        nrows_v, acc_sh.at[nsd_v.at[0, 1]], sem_s).wait()

                fetch_idx(q + 1, nsd_v)
                pltpu.async_copy(h_hbm.at[nsd_v.at[0, 0]], nrows_v, sem)

            srow_scatter(rows_v, sd_v)

    def pair(p, _):
        half(2 * p, sdA, rowsA, sdB, rowsB)
        half(2 * p + 1, sdB, rowsB, sdA, rowsA)
        return 0

    lax.fori_loop(0, (FULL + 2) // 2, pair, 0)

    pltpu.make_async_copy(rowsA, acc_sh.at[sdA.at[0, 1]], sem_s).wait()
    pltpu.make_async_copy(rowsB, acc_sh.at[sdB.at[0, 1]], sem_s).wait()
    plsc.subcore_barrier()
    _copy_out(acc_sh, acc0_o, acc1_o, c, row0, s)
    _copy_out(den_sh, den0_o, den1_o, c, row0, s)


_gat_edge_k = pl.kernel(
    _gat_edge,
    out_type=[jax.ShapeDtypeStruct((N, D), jnp.float32),
              jax.ShapeDtypeStruct((N, D), jnp.float32),
              jax.ShapeDtypeStruct((N, DEN_W), jnp.float32),
              jax.ShapeDtypeStruct((N, DEN_W), jnp.float32)],
    mesh=_sc_mesh,
    compiler_params=pltpu.CompilerParams(needs_layout_passes=False, use_tc_tiling_on_sc=False),
    scratch_types=[
        pltpu.VMEM((N,), jnp.float32),
        pltpu.VMEM((N,), jnp.float32),
        pltpu.VMEM((16,), jnp.float32),
        pltpu.VMEM((1, 2, CH), jnp.int32),
        pltpu.VMEM((1, 2, CH), jnp.int32),
        pltpu.VMEM((CH,), jnp.float32),
        pltpu.VMEM((CH, D), jnp.float32),
        pltpu.VMEM((CH, D), jnp.float32),
        pltpu.VMEM((CH, DEN_W), jnp.float32),
        pltpu.VMEM_SHARED((N, D), jnp.float32),
        pltpu.VMEM_SHARED((N, DEN_W), jnp.float32),
        pltpu.SemaphoreType.DMA,
        pltpu.SemaphoreType.DMA,
    ])


def _sage_edge(with_cnt, *refs):
    if with_cnt:
        (x_hbm, zr_hbm, z16_hbm, ones_hbm, sd_hbm,
         sacc0_o, sacc1_o, cnt0_o, cnt1_o,
         sd0, sd1, sd2, sd3, rowsA, rowsB, ones_v,
         acc_sh, cnt_sh, sem, sem_s, sem_i, sem_c) = refs
    else:
        (x_hbm, zr_hbm, sd_hbm,
         sacc0_o, sacc1_o,
         sd0, sd1, sd2, sd3, rowsA, rowsB,
         acc_sh, sem, sem_s, sem_i, sem_c) = refs
    sds = (sd0, sd1, sd2, sd3)
    c = lax.axis_index("c")
    s = lax.axis_index("s")
    wid = s * NC + c
    row0 = s * RPT
    nchunk = FULL + jnp.where(wid < EXTRA, 1, 0)

    pltpu.sync_copy(zr_hbm, rowsA)
    _init_shared(rowsA, acc_sh, row0, s)
    if with_cnt:
        pltpu.sync_copy(z16_hbm, ones_v)
        _init_shared(ones_v, cnt_sh, row0, s)
        pltpu.sync_copy(ones_hbm, ones_v)
    plsc.subcore_barrier()

    def cid(q):
        return jnp.where(q < FULL, wid + q * NW, FULL * NW + wid)

    pltpu.sync_copy(sd_hbm.at[pl.ds(cid(0), 1)], sd0)
    pltpu.async_copy(sd_hbm.at[pl.ds(cid(1), 1)], sd1, sem_i)
    pltpu.async_copy(sd_hbm.at[pl.ds(cid(2), 1)], sd2, sem_i)
    pltpu.async_copy(x_hbm.at[sd0.at[0, 0]], rowsA, sem)

    def half(q, sd_v, rows_v, nsd_v, nrows_v, fsd_v):
        @pl.when(q < nchunk)
        def _():
            pltpu.make_async_copy(x_hbm.at[sd_v.at[0, 0]], rows_v, sem).wait()

            @pl.when(q + 1 < nchunk)
            def _():
                @pl.when(q >= 1)
                def _():
                    pltpu.make_async_copy(
                        nrows_v, acc_sh.at[nsd_v.at[0, 1]], sem_s).wait()

                pltpu.make_async_copy(
                    sd_hbm.at[pl.ds(cid(q + 1), 1)], nsd_v, sem_i).wait()
                pltpu.async_copy(x_hbm.at[nsd_v.at[0, 0]], nrows_v, sem)

                @pl.when(q + 3 < nchunk)
                def _():
                    pltpu.async_copy(
                        sd_hbm.at[pl.ds(cid(q + 3), 1)], fsd_v, sem_i)

            pltpu.async_copy(rows_v, acc_sh.at[sd_v.at[0, 1]], sem_s, add=True)
            if with_cnt:
                pltpu.async_copy(ones_v, cnt_sh.at[sd_v.at[0, 1]], sem_c,
                                 add=True)

                @pl.when(q >= 2)
                def _():
                    pltpu.make_async_copy(
                        ones_v, cnt_sh.at[sd_v.at[0, 1]], sem_c).wait()

    def quad(p, _):
        q = 4 * p
        half(q, sd0, rowsA, sd1, rowsB, sd3)
        half(q + 1, sd1, rowsB, sd2, rowsA, sd0)
        half(q + 2, sd2, rowsA, sd3, rowsB, sd1)
        half(q + 3, sd3, rowsB, sd0, rowsA, sd2)
        return 0

    lax.fori_loop(0, (FULL + 4) // 4, quad, 0)

    pltpu.make_async_copy(rowsA, acc_sh.at[sd0.at[0, 1]], sem_s).wait()
    pltpu.make_async_copy(rowsB, acc_sh.at[sd1.at[0, 1]], sem_s).wait()
    if with_cnt:
        pltpu.make_async_copy(ones_v, cnt_sh.at[sd0.at[0, 1]], sem_c).wait()
        pltpu.make_async_copy(ones_v, cnt_sh.at[sd0.at[0, 1]], sem_c).wait()
    plsc.subcore_barrier()
    _copy_out(acc_sh, sacc0_o, sacc1_o, c, row0, s)
    if with_cnt:
        _copy_out(cnt_sh, cnt0_o, cnt1_o, c, row0, s)


_sc_params = pltpu.CompilerParams(needs_layout_passes=False,
                                  use_tc_tiling_on_sc=False)

_sage_cnt_k = pl.kernel(
    functools.partial(_sage_edge, True),
    out_type=[jax.ShapeDtypeStruct((N, D), jnp.float32),
              jax.ShapeDtypeStruct((N, D), jnp.float32),
              jax.ShapeDtypeStruct((N, DEN_W), jnp.float32),
              jax.ShapeDtypeStruct((N, DEN_W), jnp.float32)],
    mesh=_sc_mesh,
    compiler_params=_sc_params,
    scratch_types=[
        pltpu.VMEM((1, 2, CH), jnp.int32),
        pltpu.VMEM((1, 2, CH), jnp.int32),
        pltpu.VMEM((1, 2, CH), jnp.int32),
        pltpu.VMEM((1, 2, CH), jnp.int32),
        pltpu.VMEM((CH, D), jnp.float32),
        pltpu.VMEM((CH, D), jnp.float32),
        pltpu.VMEM((CH, DEN_W), jnp.float32),
        pltpu.VMEM_SHARED((N, D), jnp.float32),
        pltpu.VMEM_SHARED((N, DEN_W), jnp.float32),
        pltpu.SemaphoreType.DMA,
        pltpu.SemaphoreType.DMA,
        pltpu.SemaphoreType.DMA,
        pltpu.SemaphoreType.DMA,
    ])

_sage_nocnt_k = pl.kernel(
    functools.partial(_sage_edge, False),
    out_type=[jax.ShapeDtypeStruct((N, D), jnp.float32),
              jax.ShapeDtypeStruct((N, D), jnp.float32)],
    mesh=_sc_mesh,
    compiler_params=_sc_params,
    scratch_types=[
        pltpu.VMEM((1, 2, CH), jnp.int32),
        pltpu.VMEM((1, 2, CH), jnp.int32),
        pltpu.VMEM((1, 2, CH), jnp.int32),
        pltpu.VMEM((1, 2, CH), jnp.int32),
        pltpu.VMEM((CH, D), jnp.float32),
        pltpu.VMEM((CH, D), jnp.float32),
        pltpu.VMEM_SHARED((N, D), jnp.float32),
        pltpu.SemaphoreType.DMA,
        pltpu.SemaphoreType.DMA,
        pltpu.SemaphoreType.DMA,
        pltpu.SemaphoreType.DMA,
    ])

_BPW = B // NW


def _readout(x4_hbm, xv_hbm, fvm_hbm, it0_hbm, it1_hbm, it2_hbm,
             user_o, pos_o, vid_o,
             fvm_v, i0_v, i1_v, i2_v, vi_v, u_v, p_v, v_v, sem):
    c = lax.axis_index("c")
    s = lax.axis_index("s")
    wid = s * NC + c
    base = wid * _BPW
    pltpu.sync_copy(fvm_hbm, fvm_v)
    pltpu.sync_copy(it0_hbm.at[pl.ds(base, _BPW)], i0_v)
    pltpu.sync_copy(it1_hbm.at[pl.ds(base, _BPW)], i1_v)
    pltpu.sync_copy(it2_hbm.at[pl.ds(base, _BPW)], i2_v)
    for k in range(_BPW // 16):
        iv = i1_v[pl.ds(k * 16, 16)]
        vi_v[pl.ds(k * 16, 16)] = plsc.load_gather(fvm_v, [iv])
    pltpu.async_copy(x4_hbm.at[i0_v], u_v, sem).wait()
    pltpu.async_copy(x4_hbm.at[i2_v], p_v, sem).wait()
    pltpu.async_copy(xv_hbm.at[vi_v], v_v, sem).wait()
    pltpu.sync_copy(u_v, user_o.at[pl.ds(base, _BPW)])
    pltpu.sync_copy(p_v, pos_o.at[pl.ds(base, _BPW)])
    pltpu.sync_copy(v_v, vid_o.at[pl.ds(base, _BPW)])


_readout_k = pl.kernel(
    _readout,
    out_type=[jax.ShapeDtypeStruct((B, D), jnp.float32),
              jax.ShapeDtypeStruct((B, D), jnp.float32),
              jax.ShapeDtypeStruct((B, D), jnp.float32)],
    mesh=_sc_mesh,
    compiler_params=pltpu.CompilerParams(needs_layout_passes=False, use_tc_tiling_on_sc=False),
    scratch_types=[
        pltpu.VMEM((NV,), jnp.int32),
        pltpu.VMEM((_BPW,), jnp.int32),
        pltpu.VMEM((_BPW,), jnp.int32),
        pltpu.VMEM((_BPW,), jnp.int32),
        pltpu.VMEM((_BPW,), jnp.int32),
        pltpu.VMEM((_BPW, D), jnp.float32),
        pltpu.VMEM((_BPW, D), jnp.float32),
        pltpu.VMEM((_BPW, D), jnp.float32),
        pltpu.SemaphoreType.DMA,
    ])


def kernel(item, uh_edge_index, v_uh_edge_index, feature_video_mapping,
           features, u_h_embedding, W_tv, b_tv, W_g1, a1_src, a1_dst,
           W_s2l, W_s2r, W_g3, a3_src, a3_dst, W_s4l, W_s4r,
           W_uv, b_uv, W_uh, b_uh):
    sd1 = jnp.stack([v_uh_edge_index[0].reshape(NCHUNK, CH),
                     v_uh_edge_index[1].reshape(NCHUNK, CH)], axis=1)
    sd2 = jnp.stack([uh_edge_index[0].reshape(NCHUNK, CH),
                     uh_edge_index[1].reshape(NCHUNK, CH)], axis=1)
    it0 = item[:, 0].astype(jnp.int32)
    it1 = item[:, 1].astype(jnp.int32)
    it2 = item[:, 2].astype(jnp.int32)

    xv, x0 = _feat_k(features, u_h_embedding, W_tv, b_tv[None, :])

    h1, hs1, hd1, gm1 = _prep_k(x0, W_g1, a1_src[:, None], a1_dst[:, None])
    zr = jnp.zeros((CH, D), jnp.float32)
    z16 = jnp.zeros((CH, DEN_W), jnp.float32)
    ones16 = jnp.ones((CH, DEN_W), jnp.float32)
    ga0, ga1, gd0, gd1 = _gat_edge_k(h1, hs1.reshape(N), hd1.reshape(N),
                                     jnp.broadcast_to(gm1.reshape(1), (16,)),
                                     zr, z16, sd1)
    x1 = _postgat_k(ga0, ga1, gd0, gd1)

    sa0, sa1, sc0, sc1 = _sage_cnt_k(x1, zr, z16, ones16, sd2)
    x2 = _postsage_k(sa0, sa1, sc0, sc1, x1, W_s2l, W_s2r)

    h3, hs3, hd3, gm3 = _prep_k(x2, W_g3, a3_src[:, None], a3_dst[:, None])
    gb0, gb1, ge0, ge1 = _gat_edge_k(h3, hs3.reshape(N), hd3.reshape(N),
                                     jnp.broadcast_to(gm3.reshape(1), (16,)),
                                     zr, z16, sd1)
    x3 = _postgat_k(gb0, gb1, ge0, ge1)

    sb0, sb1 = _sage_nocnt_k(x3, zr, sd2)
    x4 = _postsage_k(sb0, sb1, sc0, sc1, x3, W_s4l, W_s4r)

    user, pos, vid = _readout_k(x4, xv, feature_video_mapping,
                                it0, it1, it2)
    scores = _score_k(user, pos, vid, W_uv, b_uv[None, :],
                      W_uh, b_uh[None, :])
    return scores.reshape(B)


# speedup vs baseline: 11.5776x; 1.0546x over previous
import functools

import jax
import jax.numpy as jnp
from jax import lax
from jax.experimental import pallas as pl
from jax.experimental.pallas import tpu as pltpu
from jax.experimental.pallas import tpu_sc as plsc

NU, NH, NV = 4000, 1000, 5000
NUH = NU + NH
N = NUH + NV
D, DF, E, B = 128, 512, 320000, 1024

NC, NS = 2, 16
NW = NC * NS
CH = 64
NCHUNK = E // CH
FULL = NCHUNK // NW
EXTRA = NCHUNK - FULL * NW
RPT = 624
DEN_W = 16

_HI = lax.Precision.HIGHEST


def _lrelu(x, s):
    return jnp.where(x > 0, x, x * s)


def _feat_body(feat, uh, Wtv, btv, xv_o, x0_o):
    xv = _lrelu(jnp.dot(feat[...], Wtv[...], precision=_HI) + btv[...], 0.01)
    xv_o[...] = xv
    x = jnp.concatenate([uh[...], xv], axis=0)
    nrm = jnp.sqrt(jnp.sum(x * x, axis=1, keepdims=True))
    x0_o[...] = x / jnp.maximum(nrm, 1e-12)


_feat_k = pl.pallas_call(
    _feat_body,
    out_shape=[jax.ShapeDtypeStruct((NV, D), jnp.float32),
               jax.ShapeDtypeStruct((N, D), jnp.float32)])


def _prep_body(x, Wg, a_s, a_d, h_o, hs_o, hd_o, gm_o):
    i = pl.program_id(0)
    h = jnp.dot(x[...], Wg[...], precision=_HI)
    h_o[...] = h
    hs = jnp.dot(h, a_s[...], precision=_HI)
    hd = jnp.dot(h, a_d[...], precision=_HI)
    hs_o[...] = hs
    hd_o[...] = hd

    @pl.when(i == 0)
    def _():
        gm_o[...] = jnp.full((1, 1), -jnp.inf)

    gm_o[...] = jnp.maximum(gm_o[...], jnp.max(hs))


_RB = 2000

_prep_k = pl.pallas_call(
    _prep_body,
    grid=(N // _RB,),
    in_specs=[pl.BlockSpec((_RB, D), lambda i: (i, 0)),
              pl.BlockSpec((D, D), lambda i: (0, 0)),
              pl.BlockSpec((D, 1), lambda i: (0, 0)),
              pl.BlockSpec((D, 1), lambda i: (0, 0))],
    out_specs=[pl.BlockSpec((_RB, D), lambda i: (i, 0)),
               pl.BlockSpec((_RB, 1), lambda i: (i, 0)),
               pl.BlockSpec((_RB, 1), lambda i: (i, 0)),
               pl.BlockSpec((1, 1), lambda i: (0, 0))],
    out_shape=[jax.ShapeDtypeStruct((N, D), jnp.float32),
               jax.ShapeDtypeStruct((N, 1), jnp.float32),
               jax.ShapeDtypeStruct((N, 1), jnp.float32),
               jax.ShapeDtypeStruct((1, 1), jnp.float32)])


def _postgat_body(acc0, acc1, den0, den1, x_o):
    a = acc0[...] + acc1[...]
    d = den0[:, 0] + den1[:, 0]
    x_o[...] = _lrelu(a / (d[:, None] + 1e-16), 0.01)


_postgat_k = pl.pallas_call(
    _postgat_body,
    grid=(N // _RB,),
    in_specs=[pl.BlockSpec((_RB, D), lambda i: (i, 0)),
              pl.BlockSpec((_RB, D), lambda i: (i, 0)),
              pl.BlockSpec((_RB, DEN_W), lambda i: (i, 0)),
              pl.BlockSpec((_RB, DEN_W), lambda i: (i, 0))],
    out_specs=pl.BlockSpec((_RB, D), lambda i: (i, 0)),
    out_shape=jax.ShapeDtypeStruct((N, D), jnp.float32))


def _postsage_body(sacc0, sacc1, cnt0, cnt1, x, Wl, Wr, x_o):
    s = sacc0[...] + sacc1[...]
    c = cnt0[:, 0] + cnt1[:, 0]
    mean = s / jnp.maximum(c, 1.0)[:, None]
    x_o[...] = _lrelu(jnp.dot(mean, Wl[...], precision=_HI)
                      + jnp.dot(x[...], Wr[...], precision=_HI), 0.01)


_postsage_k = pl.pallas_call(
    _postsage_body,
    grid=(N // _RB,),
    in_specs=[pl.BlockSpec((_RB, D), lambda i: (i, 0)),
              pl.BlockSpec((_RB, D), lambda i: (i, 0)),
              pl.BlockSpec((_RB, DEN_W), lambda i: (i, 0)),
              pl.BlockSpec((_RB, DEN_W), lambda i: (i, 0)),
              pl.BlockSpec((_RB, D), lambda i: (i, 0)),
              pl.BlockSpec((D, D), lambda i: (0, 0)),
              pl.BlockSpec((D, D), lambda i: (0, 0))],
    out_specs=pl.BlockSpec((_RB, D), lambda i: (i, 0)),
    out_shape=jax.ShapeDtypeStruct((N, D), jnp.float32))


def _score_body(user, pos, vid, Wuv, buv, Wuh, buh, out_o):
    usv = _lrelu(jnp.dot(jnp.concatenate([vid[...], user[...]], axis=1),
                         Wuv[...], precision=_HI) + buv[...], 0.01)
    usp = _lrelu(jnp.dot(jnp.concatenate([pos[...], user[...]], axis=1),
                         Wuh[...], precision=_HI) + buh[...], 0.01)
    out_o[...] = jnp.sum(usv * usp, axis=1, keepdims=True)


_score_k = pl.pallas_call(
    _score_body,
    out_shape=jax.ShapeDtypeStruct((B, 1), jnp.float32))


_sc_mesh = plsc.VectorSubcoreMesh(core_axis_name="c", subcore_axis_name="s")
_Z16F = functools.partial(jnp.zeros, (16,), jnp.float32)


def _init_shared(src_ref, sh_ref, row0, s):
    for j in range(RPT // CH):
        pltpu.sync_copy(src_ref, sh_ref.at[pl.ds(row0 + j * CH, CH)])
    rem = RPT - (RPT // CH) * CH
    if rem:
        pltpu.sync_copy(src_ref.at[pl.ds(0, rem)],
                        sh_ref.at[pl.ds(row0 + (RPT // CH) * CH, rem)])

    @pl.when(s == NS - 1)
    def _():
        pltpu.sync_copy(src_ref.at[pl.ds(0, N - NS * RPT)],
                        sh_ref.at[pl.ds(NS * RPT, N - NS * RPT)])


def _copy_out(sh_ref, out0_ref, out1_ref, c, row0, s):
    def emit(out_ref):
        pltpu.sync_copy(sh_ref.at[pl.ds(row0, RPT)],
                        out_ref.at[pl.ds(row0, RPT)])

        @pl.when(s == NS - 1)
        def _():
            pltpu.sync_copy(sh_ref.at[pl.ds(NS * RPT, N - NS * RPT)],
                            out_ref.at[pl.ds(NS * RPT, N - NS * RPT)])

    @pl.when(c == 0)
    def _():
        emit(out0_ref)

    @pl.when(c == 1)
    def _():
        emit(out1_ref)


def _gat_edge(h_hbm, hs_hbm, hd_hbm, g16_hbm, zr_hbm, z16_hbm, sd_hbm,
              acc0_o, acc1_o, den0_o, den1_o,
              hs_v, hd_v, g16_v, sdA, sdB, dsA, dsB, w_v,
              rowsA, rowsB, w16_v, acc_sh, den_sh, sem, sem_s, sem_i):
    c = lax.axis_index("c")
    s = lax.axis_index("s")
    wid = s * NC + c
    row0 = s * RPT
    nchunk = FULL + jnp.where(wid < EXTRA, 1, 0)

    pltpu.sync_copy(hs_hbm, hs_v)
    pltpu.sync_copy(hd_hbm, hd_v)
    pltpu.sync_copy(g16_hbm, g16_v)
    pltpu.sync_copy(zr_hbm, rowsA)
    pltpu.sync_copy(z16_hbm, w16_v)
    _init_shared(rowsA, acc_sh, row0, s)
    _init_shared(w16_v, den_sh, row0, s)
    plsc.subcore_barrier()
    g16 = g16_v[pl.ds(0, 16)]

    def cid(q):
        return jnp.where(q < FULL, wid + q * NW, FULL * NW + wid)

    def fetch_idx(q, sd_v):
        pltpu.sync_copy(sd_hbm.at[pl.ds(cid(q), 1)], sd_v)

    def wloop(sd_v):
        for k in range(CH // 16):
            sv = sd_v[0, 0, pl.ds(k * 16, 16)]
            dv = sd_v[0, 1, pl.ds(k * 16, 16)]
            hsg = plsc.load_gather(hs_v, [sv])
            hdg = plsc.load_gather(hd_v, [dv])
            mg = g16 + hdg
            mg = jnp.where(mg > 0, mg, mg * 0.2)
            a = hsg + hdg
            a = jnp.where(a > 0, a, a * 0.2)
            w_v[pl.ds(k * 16, 16)] = jnp.exp(a - mg)

    def srow(rows_v):
        for r in range(CH):
            wb = plsc.load_gather(w_v, [jnp.full((16,), r, jnp.int32)])
            w16_v[r, pl.ds(0, 16)] = wb
            for k in range(D // 16):
                rows_v[r, pl.ds(k * 16, 16)] = (
                    rows_v[r, pl.ds(k * 16, 16)] * wb)

    fetch_idx(0, sdA)
    pltpu.async_copy(sd_hbm.at[pl.ds(cid(1), 1)], sdB, sem_i)
    pltpu.async_copy(h_hbm.at[sdA.at[0, 0]], rowsA, sem)

    def half(q, sd_v, rows_v, dstS_v, nsd_v, nrows_v, ndstS_v):
        @pl.when(q < nchunk)
        def _():
            wloop(sd_v)
            for k in range(CH // 16):
                dstS_v[pl.ds(k * 16, 16)] = sd_v[0, 1, pl.ds(k * 16, 16)]
            pltpu.make_async_copy(h_hbm.at[sd_v.at[0, 0]], rows_v, sem).wait()

            @pl.when(q + 1 < nchunk)
            def _():
                @pl.when(q >= 1)
                def _():
                    pltpu.make_async_copy(
                        nrows_v, acc_sh.at[ndstS_v], sem_s).wait()

                pltpu.make_async_copy(
                    sd_hbm.at[pl.ds(cid(q + 1), 1)], nsd_v, sem_i).wait()
                pltpu.async_copy(h_hbm.at[nsd_v.at[0, 0]], nrows_v, sem)

            srow(rows_v)
            pltpu.async_copy(rows_v, acc_sh.at[dstS_v], sem_s, add=True)
            pltpu.sync_copy(w16_v, den_sh.at[dstS_v], add=True)

            @pl.when(q + 2 < nchunk)
            def _():
                pltpu.async_copy(sd_hbm.at[pl.ds(cid(q + 2), 1)], sd_v, sem_i)

    def pair(p, _):
        half(2 * p, sdA, rowsA, dsA, sdB, rowsB, dsB)
        half(2 * p + 1, sdB, rowsB, dsB, sdA, rowsA, dsA)
        return 0

    lax.fori_loop(0, (FULL + 2) // 2, pair, 0)

    pltpu.make_async_copy(rowsA, acc_sh.at[dsA], sem_s).wait()
    pltpu.make_async_copy(rowsB, acc_sh.at[dsB], sem_s).wait()
    plsc.subcore_barrier()
    _copy_out(acc_sh, acc0_o, acc1_o, c, row0, s)
    _copy_out(den_sh, den0_o, den1_o, c, row0, s)


_gat_edge_k = pl.kernel(
    _gat_edge,
    out_type=[jax.ShapeDtypeStruct((N, D), jnp.float32),
              jax.ShapeDtypeStruct((N, D), jnp.float32),
              jax.ShapeDtypeStruct((N, DEN_W), jnp.float32),
              jax.ShapeDtypeStruct((N, DEN_W), jnp.float32)],
    mesh=_sc_mesh,
    compiler_params=pltpu.CompilerParams(needs_layout_passes=False, use_tc_tiling_on_sc=False),
    scratch_types=[
        pltpu.VMEM((N,), jnp.float32),
        pltpu.VMEM((N,), jnp.float32),
        pltpu.VMEM((16,), jnp.float32),
        pltpu.VMEM((1, 2, CH), jnp.int32),
        pltpu.VMEM((1, 2, CH), jnp.int32),
        pltpu.VMEM((CH,), jnp.int32),
        pltpu.VMEM((CH,), jnp.int32),
        pltpu.VMEM((CH,), jnp.float32),
        pltpu.VMEM((CH, D), jnp.float32),
        pltpu.VMEM((CH, D), jnp.float32),
        pltpu.VMEM((CH, DEN_W), jnp.float32),
        pltpu.VMEM_SHARED((N, D), jnp.float32),
        pltpu.VMEM_SHARED((N, DEN_W), jnp.float32),
        pltpu.SemaphoreType.DMA,
        pltpu.SemaphoreType.DMA,
        pltpu.SemaphoreType.DMA,
    ])


def _sage_edge(with_cnt, *refs):
    if with_cnt:
        (x_hbm, zr_hbm, z16_hbm, ones_hbm, sd_hbm,
         sacc0_o, sacc1_o, cnt0_o, cnt1_o,
         sd0, sd1, sd2, sd3, rowsA, rowsB, ones_v,
         acc_sh, cnt_sh, sem, sem_s, sem_i, sem_c) = refs
    else:
        (x_hbm, zr_hbm, sd_hbm,
         sacc0_o, sacc1_o,
         sd0, sd1, sd2, sd3, rowsA, rowsB,
         acc_sh, sem, sem_s, sem_i, sem_c) = refs
    sds = (sd0, sd1, sd2, sd3)
    c = lax.axis_index("c")
    s = lax.axis_index("s")
    wid = s * NC + c
    row0 = s * RPT
    nchunk = FULL + jnp.where(wid < EXTRA, 1, 0)

    pltpu.sync_copy(zr_hbm, rowsA)
    _init_shared(rowsA, acc_sh, row0, s)
    if with_cnt:
        pltpu.sync_copy(z16_hbm, ones_v)
        _init_shared(ones_v, cnt_sh, row0, s)
        pltpu.sync_copy(ones_hbm, ones_v)
    plsc.subcore_barrier()

    def cid(q):
        return jnp.where(q < FULL, wid + q * NW, FULL * NW + wid)

    pltpu.sync_copy(sd_hbm.at[pl.ds(cid(0), 1)], sd0)
    pltpu.async_copy(sd_hbm.at[pl.ds(cid(1), 1)], sd1, sem_i)
    pltpu.async_copy(sd_hbm.at[pl.ds(cid(2), 1)], sd2, sem_i)
    pltpu.async_copy(x_hbm.at[sd0.at[0, 0]], rowsA, sem)

    def half(q, sd_v, rows_v, nsd_v, nrows_v, fsd_v):
        @pl.when(q < nchunk)
        def _():
            pltpu.make_async_copy(x_hbm.at[sd_v.at[0, 0]], rows_v, sem).wait()

            @pl.when(q + 1 < nchunk)
            def _():
                @pl.when(q >= 1)
                def _():
                    pltpu.make_async_copy(
                        nrows_v, acc_sh.at[nsd_v.at[0, 1]], sem_s).wait()

                pltpu.make_async_copy(
                    sd_hbm.at[pl.ds(cid(q + 1), 1)], nsd_v, sem_i).wait()
                pltpu.async_copy(x_hbm.at[nsd_v.at[0, 0]], nrows_v, sem)

                @pl.when(q + 3 < nchunk)
                def _():
                    pltpu.async_copy(
                        sd_hbm.at[pl.ds(cid(q + 3), 1)], fsd_v, sem_i)

            pltpu.async_copy(rows_v, acc_sh.at[sd_v.at[0, 1]], sem_s, add=True)
            if with_cnt:
                pltpu.async_copy(ones_v, cnt_sh.at[sd_v.at[0, 1]], sem_c,
                                 add=True)

                @pl.when(q >= 2)
                def _():
                    pltpu.make_async_copy(
                        ones_v, cnt_sh.at[sd_v.at[0, 1]], sem_c).wait()

    def quad(p, _):
        q = 4 * p
        half(q, sd0, rowsA, sd1, rowsB, sd3)
        half(q + 1, sd1, rowsB, sd2, rowsA, sd0)
        half(q + 2, sd2, rowsA, sd3, rowsB, sd1)
        half(q + 3, sd3, rowsB, sd0, rowsA, sd2)
        return 0

    lax.fori_loop(0, (FULL + 4) // 4, quad, 0)

    pltpu.make_async_copy(rowsA, acc_sh.at[sd0.at[0, 1]], sem_s).wait()
    pltpu.make_async_copy(rowsB, acc_sh.at[sd1.at[0, 1]], sem_s).wait()
    if with_cnt:
        pltpu.make_async_copy(ones_v, cnt_sh.at[sd0.at[0, 1]], sem_c).wait()
        pltpu.make_async_copy(ones_v, cnt_sh.at[sd0.at[0, 1]], sem_c).wait()
    plsc.subcore_barrier()
    _copy_out(acc_sh, sacc0_o, sacc1_o, c, row0, s)
    if with_cnt:
        _copy_out(cnt_sh, cnt0_o, cnt1_o, c, row0, s)


_sc_params = pltpu.CompilerParams(needs_layout_passes=False,
                                  use_tc_tiling_on_sc=False)

_sage_cnt_k = pl.kernel(
    functools.partial(_sage_edge, True),
    out_type=[jax.ShapeDtypeStruct((N, D), jnp.float32),
              jax.ShapeDtypeStruct((N, D), jnp.float32),
              jax.ShapeDtypeStruct((N, DEN_W), jnp.float32),
              jax.ShapeDtypeStruct((N, DEN_W), jnp.float32)],
    mesh=_sc_mesh,
    compiler_params=_sc_params,
    scratch_types=[
        pltpu.VMEM((1, 2, CH), jnp.int32),
        pltpu.VMEM((1, 2, CH), jnp.int32),
        pltpu.VMEM((1, 2, CH), jnp.int32),
        pltpu.VMEM((1, 2, CH), jnp.int32),
        pltpu.VMEM((CH, D), jnp.float32),
        pltpu.VMEM((CH, D), jnp.float32),
        pltpu.VMEM((CH, DEN_W), jnp.float32),
        pltpu.VMEM_SHARED((N, D), jnp.float32),
        pltpu.VMEM_SHARED((N, DEN_W), jnp.float32),
        pltpu.SemaphoreType.DMA,
        pltpu.SemaphoreType.DMA,
        pltpu.SemaphoreType.DMA,
        pltpu.SemaphoreType.DMA,
    ])

_sage_nocnt_k = pl.kernel(
    functools.partial(_sage_edge, False),
    out_type=[jax.ShapeDtypeStruct((N, D), jnp.float32),
              jax.ShapeDtypeStruct((N, D), jnp.float32)],
    mesh=_sc_mesh,
    compiler_params=_sc_params,
    scratch_types=[
        pltpu.VMEM((1, 2, CH), jnp.int32),
        pltpu.VMEM((1, 2, CH), jnp.int32),
        pltpu.VMEM((1, 2, CH), jnp.int32),
        pltpu.VMEM((1, 2, CH), jnp.int32),
        pltpu.VMEM((CH, D), jnp.float32),
        pltpu.VMEM((CH, D), jnp.float32),
        pltpu.VMEM_SHARED((N, D), jnp.float32),
        pltpu.SemaphoreType.DMA,
        pltpu.SemaphoreType.DMA,
        pltpu.SemaphoreType.DMA,
        pltpu.SemaphoreType.DMA,
    ])

_BPW = B // NW


def _readout(x4_hbm, xv_hbm, fvm_hbm, it0_hbm, it1_hbm, it2_hbm,
             user_o, pos_o, vid_o,
             fvm_v, i0_v, i1_v, i2_v, vi_v, u_v, p_v, v_v, sem):
    c = lax.axis_index("c")
    s = lax.axis_index("s")
    wid = s * NC + c
    base = wid * _BPW
    pltpu.sync_copy(fvm_hbm, fvm_v)
    pltpu.sync_copy(it0_hbm.at[pl.ds(base, _BPW)], i0_v)
    pltpu.sync_copy(it1_hbm.at[pl.ds(base, _BPW)], i1_v)
    pltpu.sync_copy(it2_hbm.at[pl.ds(base, _BPW)], i2_v)
    for k in range(_BPW // 16):
        iv = i1_v[pl.ds(k * 16, 16)]
        vi_v[pl.ds(k * 16, 16)] = plsc.load_gather(fvm_v, [iv])
    pltpu.async_copy(x4_hbm.at[i0_v], u_v, sem).wait()
    pltpu.async_copy(x4_hbm.at[i2_v], p_v, sem).wait()
    pltpu.async_copy(xv_hbm.at[vi_v], v_v, sem).wait()
    pltpu.sync_copy(u_v, user_o.at[pl.ds(base, _BPW)])
    pltpu.sync_copy(p_v, pos_o.at[pl.ds(base, _BPW)])
    pltpu.sync_copy(v_v, vid_o.at[pl.ds(base, _BPW)])


_readout_k = pl.kernel(
    _readout,
    out_type=[jax.ShapeDtypeStruct((B, D), jnp.float32),
              jax.ShapeDtypeStruct((B, D), jnp.float32),
              jax.ShapeDtypeStruct((B, D), jnp.float32)],
    mesh=_sc_mesh,
    compiler_params=pltpu.CompilerParams(needs_layout_passes=False, use_tc_tiling_on_sc=False),
    scratch_types=[
        pltpu.VMEM((NV,), jnp.int32),
        pltpu.VMEM((_BPW,), jnp.int32),
        pltpu.VMEM((_BPW,), jnp.int32),
        pltpu.VMEM((_BPW,), jnp.int32),
        pltpu.VMEM((_BPW,), jnp.int32),
        pltpu.VMEM((_BPW, D), jnp.float32),
        pltpu.VMEM((_BPW, D), jnp.float32),
        pltpu.VMEM((_BPW, D), jnp.float32),
        pltpu.SemaphoreType.DMA,
    ])


def kernel(item, uh_edge_index, v_uh_edge_index, feature_video_mapping,
           features, u_h_embedding, W_tv, b_tv, W_g1, a1_src, a1_dst,
           W_s2l, W_s2r, W_g3, a3_src, a3_dst, W_s4l, W_s4r,
           W_uv, b_uv, W_uh, b_uh):
    sd1 = jnp.stack([v_uh_edge_index[0].reshape(NCHUNK, CH),
                     v_uh_edge_index[1].reshape(NCHUNK, CH)], axis=1)
    sd2 = jnp.stack([uh_edge_index[0].reshape(NCHUNK, CH),
                     uh_edge_index[1].reshape(NCHUNK, CH)], axis=1)
    it0 = item[:, 0].astype(jnp.int32)
    it1 = item[:, 1].astype(jnp.int32)
    it2 = item[:, 2].astype(jnp.int32)

    xv, x0 = _feat_k(features, u_h_embedding, W_tv, b_tv[None, :])

    h1, hs1, hd1, gm1 = _prep_k(x0, W_g1, a1_src[:, None], a1_dst[:, None])
    zr = jnp.zeros((CH, D), jnp.float32)
    z16 = jnp.zeros((CH, DEN_W), jnp.float32)
    ones16 = jnp.ones((CH, DEN_W), jnp.float32)
    ga0, ga1, gd0, gd1 = _gat_edge_k(h1, hs1.reshape(N), hd1.reshape(N),
                                     jnp.broadcast_to(gm1.reshape(1), (16,)),
                                     zr, z16, sd1)
    x1 = _postgat_k(ga0, ga1, gd0, gd1)

    sa0, sa1, sc0, sc1 = _sage_cnt_k(x1, zr, z16, ones16, sd2)
    x2 = _postsage_k(sa0, sa1, sc0, sc1, x1, W_s2l, W_s2r)

    h3, hs3, hd3, gm3 = _prep_k(x2, W_g3, a3_src[:, None], a3_dst[:, None])
    gb0, gb1, ge0, ge1 = _gat_edge_k(h3, hs3.reshape(N), hd3.reshape(N),
                                     jnp.broadcast_to(gm3.reshape(1), (16,)),
                                     zr, z16, sd1)
    x3 = _postgat_k(gb0, gb1, ge0, ge1)

    sb0, sb1 = _sage_nocnt_k(x3, zr, sd2)
    x4 = _postsage_k(sb0, sb1, sc0, sc1, x3, W_s4l, W_s4r)

    user, pos, vid = _readout_k(x4, xv, feature_video_mapping,
                                it0, it1, it2)
    scores = _score_k(user, pos, vid, W_uv, b_uv[None, :],
                      W_uh, b_uh[None, :])
    return scores.reshape(B)
